# baseline (speedup 1.0000x reference)
import functools
import jax
import jax.numpy as jnp
from jax import lax
from jax.experimental import pallas as pl
from jax.experimental.pallas import tpu as pltpu
from jax.experimental.pallas import tpu_sc as plsc

N = 10000
E = 320000
H = 128
G = 64
C = 10
EPS = 1e-5

NC = 2
NS = 16
NW = NC * NS
EPW = E // NW
KE = 112
NCE = 91
NPAIR = (NCE - 1) // 2
EPAD = NW * NCE * KE
KD = 2000
NCD = EPW // KD
NA = 10240
TR = NA // NS
NP = 10752
CH = 512


def _sc_mesh():
    return plsc.VectorSubcoreMesh(
        core_axis_name="c", subcore_axis_name="s",
        num_cores=NC, num_subcores=NS)


def _sc_deg_body(c_hbm, out_hbm, cidx, ones_v, acc):
    core = lax.axis_index("c")
    sid = lax.axis_index("s")
    w = core * NS + sid

    def zrow(i, _):
        ones_v[i, :] = jnp.zeros((16,), jnp.float32)
        return 0
    lax.fori_loop(0, TR, zrow, 0)
    pltpu.sync_copy(ones_v.at[pl.ds(0, TR)], acc.at[pl.ds(sid * TR, TR)])

    def orow(i, _):
        ones_v[i, :] = jnp.ones((16,), jnp.float32)
        return 0
    lax.fori_loop(0, KD, orow, 0)
    plsc.subcore_barrier()

    base = w * EPW

    def body(j, _):
        pltpu.sync_copy(c_hbm.at[pl.ds(base + j * KD, KD)], cidx)
        pltpu.sync_copy(ones_v, acc.at[cidx], add=True)
        return 0
    lax.fori_loop(0, NCD, body, 0)
    plsc.subcore_barrier()
    pltpu.sync_copy(acc.at[pl.ds(sid * TR, TR)],
                    out_hbm.at[core, pl.ds(sid * TR, TR)])


def _sc_edge_body(r_hbm, c_hbm, g_hbm, out_hbm, ridx, cidx,
                  rows0, rows1, acc, sem0, sem1):
    core = lax.axis_index("c")
    sid = lax.axis_index("s")
    w = core * NS + sid

    def zrow(i, _):
        for j in range(H // 16):
            rows0[i, pl.ds(j * 16, 16)] = jnp.zeros((16,), jnp.float32)
        return 0
    lax.fori_loop(0, KE, zrow, 0)
    off = 0
    while off < TR:
        step = min(KE, TR - off)
        pltpu.sync_copy(rows0.at[pl.ds(0, step)],
                        acc.at[pl.ds(sid * TR + off, step)])
        off += step
    plsc.subcore_barrier()

    pltpu.sync_copy(r_hbm.at[w, pl.ds(0, NCE)], ridx)
    pltpu.sync_copy(c_hbm.at[w, pl.ds(0, NCE)], cidx)

    def gather(j, buf, sem):
        pltpu.async_copy(g_hbm.at[ridx.at[j]], buf, sem)

    def gwait(buf, sem):
        pltpu.make_async_copy(g_hbm.at[ridx.at[0]], buf, sem).wait()

    gather(0, rows0, sem0)

    def body(p, _):
        j0 = 2 * p
        gwait(rows0, sem0)
        gather(j0 + 1, rows1, sem1)
        pltpu.sync_copy(rows0, acc.at[cidx.at[j0]], add=True)
        gwait(rows1, sem1)
        gather(j0 + 2, rows0, sem0)
        pltpu.sync_copy(rows1, acc.at[cidx.at[j0 + 1]], add=True)
        return 0
    lax.fori_loop(0, NPAIR, body, 0)
    gwait(rows0, sem0)
    pltpu.sync_copy(rows0, acc.at[cidx.at[NCE - 1]], add=True)
    plsc.subcore_barrier()
    pltpu.sync_copy(acc.at[pl.ds(sid * TR, TR)],
                    out_hbm.at[core, pl.ds(sid * TR, TR)])


@functools.cache
def _get_sc_deg():
    return pl.kernel(
        _sc_deg_body,
        out_type=jax.ShapeDtypeStruct((NC, NA, 16), jnp.float32),
        mesh=_sc_mesh(),
        compiler_params=pltpu.CompilerParams(use_tc_tiling_on_sc=False),
        scratch_types=[
            pltpu.VMEM((KD,), jnp.int32),
            pltpu.VMEM((KD, 16), jnp.float32),
            pltpu.VMEM_SHARED((NA, 16), jnp.float32),
        ],
    )


@functools.cache
def _get_sc_edge():
    return pl.kernel(
        _sc_edge_body,
        out_type=jax.ShapeDtypeStruct((NC, NA, H), jnp.float32),
        mesh=_sc_mesh(),
        compiler_params=pltpu.CompilerParams(use_tc_tiling_on_sc=False),
        scratch_types=[
            pltpu.VMEM((NCE, KE), jnp.int32),
            pltpu.VMEM((NCE, KE), jnp.int32),
            pltpu.VMEM((KE, H), jnp.float32),
            pltpu.VMEM((KE, H), jnp.float32),
            pltpu.VMEM_SHARED((NA, H), jnp.float32),
            pltpu.SemaphoreType.DMA,
            pltpu.SemaphoreType.DMA,
        ],
    )


def _sc_deg(c):
    return _get_sc_deg()(c)


def _sc_edge(r3, c3, g):
    return _get_sc_edge()(r3, c3, g)


_HI = lax.Precision.HIGHEST


def _dinv_from(dp_ref):
    d0 = dp_ref[0]
    d1 = dp_ref[1]
    return lax.rsqrt(1.0 + d0[:N, 0:1] + d1[:N, 0:1])


def _tc_g1_body(x_ref, w_ref, dp_ref, g_ref):
    dinv = _dinv_from(dp_ref)
    g_ref[...] = dinv * lax.dot(x_ref[...], w_ref[...], precision=_HI)


def _mk_g1(interpret=False):
    return pl.pallas_call(
        _tc_g1_body,
        out_shape=jax.ShapeDtypeStruct((N, H), jnp.float32),
        interpret=interpret,
    )


_tc_g1 = _mk_g1()


def _leaky(v):
    return jnp.where(v > 0, v, 0.01 * v)


RB = 1000
NRB = N // RB


def _post_body(is_last, dp_ref, accp_ref, g_ref, b_ref, nw_ref, nb_ref,
               batch_ref, starts_ref, xr_ref, wa_ref, wb_ref, wc_ref,
               ba_ref, bb_ref, bc_ref, o0_ref, o1_ref, hs_ref, gmp_ref):

    def _blk_dinv(i):
        r0 = pl.multiple_of(i * RB, 8)
        d0 = dp_ref[0, pl.ds(r0, RB), :]
        d1 = dp_ref[1, pl.ds(r0, RB), :]
        return lax.rsqrt(1.0 + d0[:, 0:1] + d1[:, 0:1])

    def _blk_oh(i):
        r0 = pl.multiple_of(i * RB, 8)
        bb = batch_ref[pl.ds(r0, RB), :]
        return (bb == lax.broadcasted_iota(jnp.int32, (RB, G), 1)
                ).astype(jnp.float32)

    def blk1(i, carry):
        s1, s2, cnt = carry
        r0 = pl.multiple_of(i * RB, 8)
        a = accp_ref[0, pl.ds(r0, RB), :] + accp_ref[1, pl.ds(r0, RB), :]
        out = _blk_dinv(i) * (a + g_ref[pl.ds(r0, RB), :]) + b_ref[...]
        hs_ref[pl.ds(r0, RB), :] = out
        oh = _blk_oh(i)
        dn = (((0,), (0,)), ((), ()))
        s1 = s1 + lax.dot_general(oh, out, dn, precision=_HI)
        s2 = s2 + lax.dot_general(oh, out * out, dn, precision=_HI)
        cnt = cnt + lax.dot_general(oh, jnp.ones((RB, 1), jnp.float32), dn,
                                    precision=_HI)
        return s1, s2, cnt

    z = jnp.zeros((G, H), jnp.float32)
    s1, s2, cnt = lax.fori_loop(
        0, NRB, blk1, (z, z, jnp.zeros((G, 1), jnp.float32)))

    cntf = jnp.maximum(cnt * H, 1.0)
    mean = jnp.sum(s1, axis=1, keepdims=True) / cntf
    sq = jnp.sum(s2, axis=1, keepdims=True) / cntf
    var = jnp.maximum(sq - mean * mean, 0.0)
    rsig = lax.rsqrt(var + EPS)
    stats = jnp.concatenate([mean, rsig], axis=1)

    def blk2(i, gs):
        r0 = pl.multiple_of(i * RB, 8)
        out = hs_ref[pl.ds(r0, RB), :]
        oh = _blk_oh(i)
        nst = lax.dot_general(oh, stats, (((1,), (0,)), ((), ())),
                              precision=_HI)
        xn = (out - nst[:, 0:1]) * nst[:, 1:2] * nw_ref[...] + nb_ref[...]
        h = _leaky(xn)
        hs_ref[pl.ds(r0, RB), :] = h
        if not is_last:
            o0_ref[pl.ds(r0, RB), :] = _blk_dinv(i) * lax.dot(
                h, wa_ref[...], precision=_HI)
        return gs + lax.dot_general(oh, h, (((0,), (0,)), ((), ())),
                                    precision=_HI)

    gs = lax.fori_loop(0, NRB, blk2, z)
    gap = gs / jnp.maximum(cnt, 1.0)

    def graph_body(gi, _):
        start = starts_ref[gi]
        num = starts_ref[gi + 1] - start
        nch = (num + CH - 1) // CH

        def chunk_body(kk, m):
            astart = ((start + kk * CH) // 8) * 8
            astart = pl.multiple_of(astart, 8)
            win = hs_ref[pl.ds(astart, CH + 8), :]
            ridx = astart + lax.broadcasted_iota(jnp.int32, (CH + 8, 1), 0)
            lo = start + kk * CH
            hi = start + jnp.minimum((kk + 1) * CH, num)
            valid = (ridx >= lo) & (ridx < hi)
            vals = jnp.where(valid, win, -jnp.inf)
            return jnp.maximum(m, jnp.max(vals, axis=0, keepdims=True))

        m = lax.fori_loop(0, nch, chunk_body, jnp.full((1, H), -jnp.inf))
        m = jnp.where(num > 0, m, jnp.zeros((1, H), jnp.float32))
        gmp_ref[pl.ds(gi, 1), :] = m
        return 0
    lax.fori_loop(0, G, graph_body, 0)

    xp = jnp.concatenate([gmp_ref[...], gap], axis=1) + xr_ref[...]

    if not is_last:
        o1_ref[...] = xp
    else:
        z1 = _leaky(lax.dot(xp, wa_ref[...], precision=_HI) + ba_ref[...])
        z2 = _leaky(lax.dot(z1, wb_ref[...], precision=_HI) + bb_ref[...])
        logits = lax.dot(z2, wc_ref[...], precision=_HI) + bc_ref[...]
        colid = lax.broadcasted_iota(jnp.int32, (G, H), 1)
        lmask = colid < C
        lw = jnp.where(lmask, logits, -jnp.inf)
        mx = jnp.max(lw, axis=1, keepdims=True)
        ex = jnp.where(lmask, jnp.exp(lw - mx), 0.0)
        lse = jnp.log(jnp.sum(ex, axis=1, keepdims=True)) + mx
        o0_ref[...] = logits - lse


def _mk_post(is_last, interpret=False):
    in_specs = [pl.BlockSpec(memory_space=pltpu.VMEM) for _ in range(15)]
    in_specs[7] = pl.BlockSpec(memory_space=pltpu.SMEM)
    if is_last:
        out_shape = [jax.ShapeDtypeStruct((G, H), jnp.float32),
                     jax.ShapeDtypeStruct((G, H), jnp.float32)]
    else:
        out_shape = [jax.ShapeDtypeStruct((N, H), jnp.float32),
                     jax.ShapeDtypeStruct((G, 2 * H), jnp.float32)]
    return pl.pallas_call(
        functools.partial(_post_body, is_last),
        out_shape=out_shape,
        in_specs=in_specs,
        scratch_shapes=[pltpu.VMEM((NP, H), jnp.float32),
                        pltpu.VMEM((G, H), jnp.float32)],
        interpret=interpret,
    )


_post_mid = _mk_post(False)
_post_last = _mk_post(True)


def kernel(x, edge_index, batch, conv1_W, conv1_b, conv2_W, conv2_b,
           conv3_W, conv3_b, n1_w, n1_b, n2_w, n2_b, n3_w, n3_b,
           lin1_W, lin1_b, lin2_W, lin2_b, lin3_W, lin3_b):
    r = edge_index[0]
    c = edge_index[1]
    rp = jnp.concatenate(
        [r, jnp.zeros((EPAD - E,), jnp.int32)])
    cp = jnp.concatenate(
        [c, jnp.full((EPAD - E,), NA - 1, jnp.int32)])
    r3 = rp.reshape(NW, NCE, KE)
    c3 = cp.reshape(NW, NCE, KE)
    batch2 = batch.astype(jnp.int32).reshape(N, 1)
    starts = jnp.searchsorted(
        batch, jnp.arange(G + 1, dtype=batch.dtype)).astype(jnp.int32)

    degp = _sc_deg(c)

    g1 = _tc_g1(x, conv1_W, degp)
    acc1 = _sc_edge(r3, c3, g1)

    xr0 = jnp.zeros((G, 2 * H), jnp.float32)
    b1 = conv1_b.reshape(1, H)
    dummy_b = jnp.zeros((1, H), jnp.float32)
    dummy_w = jnp.zeros((H, H), jnp.float32)
    g2, xr1 = _post_mid(degp, acc1, g1, b1, n1_w.reshape(1, H),
                        n1_b.reshape(1, H), batch2, starts, xr0,
                        conv2_W, dummy_w, dummy_w, dummy_b, dummy_b, dummy_b)
    acc2 = _sc_edge(r3, c3, g2)

    g3, xr2 = _post_mid(degp, acc2, g2, conv2_b.reshape(1, H),
                        n2_w.reshape(1, H), n2_b.reshape(1, H), batch2,
                        starts, xr1, conv3_W, dummy_w, dummy_w,
                        dummy_b, dummy_b, dummy_b)
    acc3 = _sc_edge(r3, c3, g3)

    lin3p = jnp.zeros((H // 2, H), jnp.float32).at[:, :C].set(lin3_W)
    lb3p = jnp.zeros((1, H), jnp.float32).at[:, :C].set(lin3_b.reshape(1, C))
    lb2p = lin2_b.reshape(1, H // 2)
    outp, _ = _post_last(degp, acc3, g3, conv3_b.reshape(1, H),
                         n3_w.reshape(1, H), n3_b.reshape(1, H), batch2,
                         starts, xr2, lin1_W, lin2_W, lin3p,
                         lin1_b.reshape(1, H), lb2p, lb3p)
    return outp[:, :C]


# speedup vs baseline: 2.0577x; 2.0577x over previous
import functools
import jax
import jax.numpy as jnp
from jax import lax
from jax.experimental import pallas as pl
from jax.experimental.pallas import tpu as pltpu
from jax.experimental.pallas import tpu_sc as plsc

N = 10000
E = 320000
H = 128
G = 64
C = 10
EPS = 1e-5

NC = 2
NS = 16
NW = NC * NS
EPW = E // NW
KE = 80
NCE = 125
NPAIR = (NCE - 1) // 2
EPAD = NW * NCE * KE
KD = 2000
NCD = EPW // KD
NA = 10240
TR = NA // NS
NP = 10752
CH = 512


def _sc_mesh():
    return plsc.VectorSubcoreMesh(
        core_axis_name="c", subcore_axis_name="s",
        num_cores=NC, num_subcores=NS)


def _sc_deg_body(c_hbm, out_hbm, cidx, ones_v, acc):
    core = lax.axis_index("c")
    sid = lax.axis_index("s")
    w = core * NS + sid

    def zrow(i, _):
        ones_v[i, :] = jnp.zeros((16,), jnp.float32)
        return 0
    lax.fori_loop(0, TR, zrow, 0)
    pltpu.sync_copy(ones_v.at[pl.ds(0, TR)], acc.at[pl.ds(sid * TR, TR)])

    def orow(i, _):
        ones_v[i, :] = jnp.ones((16,), jnp.float32)
        return 0
    lax.fori_loop(0, KD, orow, 0)
    plsc.subcore_barrier()

    base = w * EPW

    def body(j, _):
        pltpu.sync_copy(c_hbm.at[pl.ds(base + j * KD, KD)], cidx)
        pltpu.sync_copy(ones_v, acc.at[cidx], add=True)
        return 0
    lax.fori_loop(0, NCD, body, 0)
    plsc.subcore_barrier()
    pltpu.sync_copy(acc.at[pl.ds(sid * TR, TR)],
                    out_hbm.at[core, pl.ds(sid * TR, TR)])


def _sc_edge_body(r_hbm, c_hbm, g_hbm, out_hbm, ridx, cidx,
                  rows0, rows1, acc, sem0, sem1):
    core = lax.axis_index("c")
    sid = lax.axis_index("s")
    w = core * NS + sid

    def zrow(i, _):
        for j in range(H // 16):
            rows0[i, pl.ds(j * 16, 16)] = jnp.zeros((16,), jnp.float32)
        return 0
    lax.fori_loop(0, KE, zrow, 0)
    off = 0
    while off < TR:
        step = min(KE, TR - off)
        pltpu.sync_copy(rows0.at[pl.ds(0, step)],
                        acc.at[pl.ds(sid * TR + off, step)])
        off += step
    plsc.subcore_barrier()

    pltpu.sync_copy(r_hbm.at[w, pl.ds(0, NCE)], ridx)
    pltpu.sync_copy(c_hbm.at[w, pl.ds(0, NCE)], cidx)

    def gather(j, buf, sem):
        pltpu.async_copy(g_hbm.at[ridx.at[j]], buf, sem)

    def gwait(buf, sem):
        pltpu.make_async_copy(g_hbm.at[ridx.at[0]], buf, sem).wait()

    gather(0, rows0, sem0)

    def body(p, _):
        j0 = 2 * p
        gwait(rows0, sem0)
        gather(j0 + 1, rows1, sem1)
        pltpu.sync_copy(rows0, acc.at[cidx.at[j0]], add=True)
        gwait(rows1, sem1)
        gather(j0 + 2, rows0, sem0)
        pltpu.sync_copy(rows1, acc.at[cidx.at[j0 + 1]], add=True)
        return 0
    lax.fori_loop(0, NPAIR, body, 0)
    gwait(rows0, sem0)
    pltpu.sync_copy(rows0, acc.at[cidx.at[NCE - 1]], add=True)
    plsc.subcore_barrier()
    pltpu.sync_copy(acc.at[pl.ds(sid * TR, TR)],
                    out_hbm.at[core, pl.ds(sid * TR, TR)])


@functools.cache
def _get_sc_deg():
    return pl.kernel(
        _sc_deg_body,
        out_type=jax.ShapeDtypeStruct((NC, NA, 16), jnp.float32),
        mesh=_sc_mesh(),
        compiler_params=pltpu.CompilerParams(use_tc_tiling_on_sc=False),
        scratch_types=[
            pltpu.VMEM((KD,), jnp.int32),
            pltpu.VMEM((KD, 16), jnp.float32),
            pltpu.VMEM_SHARED((NA, 16), jnp.float32),
        ],
    )


@functools.cache
def _get_sc_edge():
    return pl.kernel(
        _sc_edge_body,
        out_type=jax.ShapeDtypeStruct((NC, NA, H), jnp.float32),
        mesh=_sc_mesh(),
        compiler_params=pltpu.CompilerParams(use_tc_tiling_on_sc=False),
        scratch_types=[
            pltpu.VMEM((NCE, KE), jnp.int32),
            pltpu.VMEM((NCE, KE), jnp.int32),
            pltpu.VMEM((KE, H), jnp.float32),
            pltpu.VMEM((KE, H), jnp.float32),
            pltpu.VMEM_SHARED((NA, H), jnp.float32),
            pltpu.SemaphoreType.DMA,
            pltpu.SemaphoreType.DMA,
        ],
    )


def _sc_deg(c):
    return _get_sc_deg()(c)


def _sc_edge(r3, c3, g):
    return _get_sc_edge()(r3, c3, g)


_HI = lax.Precision.HIGHEST


def _dinv_from(dp_ref):
    d0 = dp_ref[0]
    d1 = dp_ref[1]
    return lax.rsqrt(1.0 + d0[:N, 0:1] + d1[:N, 0:1])


def _tc_g1_body(x_ref, w_ref, dp_ref, g_ref):
    dinv = _dinv_from(dp_ref)
    g_ref[...] = dinv * lax.dot(x_ref[...], w_ref[...], precision=_HI)


def _mk_g1(interpret=False):
    return pl.pallas_call(
        _tc_g1_body,
        out_shape=jax.ShapeDtypeStruct((N, H), jnp.float32),
        interpret=interpret,
    )


_tc_g1 = _mk_g1()


def _leaky(v):
    return jnp.where(v > 0, v, 0.01 * v)


RB = 1000
NRB = N // RB


def _post_body(is_last, dp_ref, accp_ref, g_ref, b_ref, nw_ref, nb_ref,
               batch_ref, starts_ref, xr_ref, wa_ref, wb_ref, wc_ref,
               ba_ref, bb_ref, bc_ref, o0_ref, o1_ref, hs_ref, gmp_ref):

    def _blk_dinv(i):
        r0 = pl.multiple_of(i * RB, 8)
        d0 = dp_ref[0, pl.ds(r0, RB), :]
        d1 = dp_ref[1, pl.ds(r0, RB), :]
        return lax.rsqrt(1.0 + d0[:, 0:1] + d1[:, 0:1])

    def _blk_oh(i):
        r0 = pl.multiple_of(i * RB, 8)
        bb = batch_ref[pl.ds(r0, RB), :]
        return (bb == lax.broadcasted_iota(jnp.int32, (RB, G), 1)
                ).astype(jnp.float32)

    def blk1(i, carry):
        s1, s2, cnt = carry
        r0 = pl.multiple_of(i * RB, 8)
        a = accp_ref[0, pl.ds(r0, RB), :] + accp_ref[1, pl.ds(r0, RB), :]
        out = _blk_dinv(i) * (a + g_ref[pl.ds(r0, RB), :]) + b_ref[...]
        hs_ref[pl.ds(r0, RB), :] = out
        oh = _blk_oh(i)
        dn = (((0,), (0,)), ((), ()))
        s1 = s1 + lax.dot_general(oh, out, dn, precision=_HI)
        s2 = s2 + lax.dot_general(oh, out * out, dn, precision=_HI)
        cnt = cnt + lax.dot_general(oh, jnp.ones((RB, 1), jnp.float32), dn,
                                    precision=_HI)
        return s1, s2, cnt

    z = jnp.zeros((G, H), jnp.float32)
    s1, s2, cnt = lax.fori_loop(
        0, NRB, blk1, (z, z, jnp.zeros((G, 1), jnp.float32)))

    cntf = jnp.maximum(cnt * H, 1.0)
    mean = jnp.sum(s1, axis=1, keepdims=True) / cntf
    sq = jnp.sum(s2, axis=1, keepdims=True) / cntf
    var = jnp.maximum(sq - mean * mean, 0.0)
    rsig = lax.rsqrt(var + EPS)
    stats = jnp.concatenate([mean, rsig], axis=1)

    def blk2(i, gs):
        r0 = pl.multiple_of(i * RB, 8)
        out = hs_ref[pl.ds(r0, RB), :]
        oh = _blk_oh(i)
        nst = lax.dot_general(oh, stats, (((1,), (0,)), ((), ())),
                              precision=_HI)
        xn = (out - nst[:, 0:1]) * nst[:, 1:2] * nw_ref[...] + nb_ref[...]
        h = _leaky(xn)
        hs_ref[pl.ds(r0, RB), :] = h
        if not is_last:
            o0_ref[pl.ds(r0, RB), :] = _blk_dinv(i) * lax.dot(
                h, wa_ref[...], precision=_HI)
        return gs + lax.dot_general(oh, h, (((0,), (0,)), ((), ())),
                                    precision=_HI)

    gs = lax.fori_loop(0, NRB, blk2, z)
    gap = gs / jnp.maximum(cnt, 1.0)

    def graph_body(gi, _):
        start = starts_ref[gi]
        num = starts_ref[gi + 1] - start
        nch = (num + CH - 1) // CH

        def chunk_body(kk, m):
            astart = ((start + kk * CH) // 8) * 8
            astart = pl.multiple_of(astart, 8)
            win = hs_ref[pl.ds(astart, CH + 8), :]
            ridx = astart + lax.broadcasted_iota(jnp.int32, (CH + 8, 1), 0)
            lo = start + kk * CH
            hi = start + jnp.minimum((kk + 1) * CH, num)
            valid = (ridx >= lo) & (ridx < hi)
            vals = jnp.where(valid, win, -jnp.inf)
            return jnp.maximum(m, jnp.max(vals, axis=0, keepdims=True))

        m = lax.fori_loop(0, nch, chunk_body, jnp.full((1, H), -jnp.inf))
        m = jnp.where(num > 0, m, jnp.zeros((1, H), jnp.float32))
        gmp_ref[pl.ds(gi, 1), :] = m
        return 0
    lax.fori_loop(0, G, graph_body, 0)

    xp = jnp.concatenate([gmp_ref[...], gap], axis=1) + xr_ref[...]

    if not is_last:
        o1_ref[...] = xp
    else:
        z1 = _leaky(lax.dot(xp, wa_ref[...], precision=_HI) + ba_ref[...])
        z2 = _leaky(lax.dot(z1, wb_ref[...], precision=_HI) + bb_ref[...])
        logits = lax.dot(z2, wc_ref[...], precision=_HI) + bc_ref[...]
        colid = lax.broadcasted_iota(jnp.int32, (G, H), 1)
        lmask = colid < C
        lw = jnp.where(lmask, logits, -jnp.inf)
        mx = jnp.max(lw, axis=1, keepdims=True)
        ex = jnp.where(lmask, jnp.exp(lw - mx), 0.0)
        lse = jnp.log(jnp.sum(ex, axis=1, keepdims=True)) + mx
        o0_ref[...] = logits - lse


def _mk_post(is_last, interpret=False):
    in_specs = [pl.BlockSpec(memory_space=pltpu.VMEM) for _ in range(15)]
    in_specs[7] = pl.BlockSpec(memory_space=pltpu.SMEM)
    if is_last:
        out_shape = [jax.ShapeDtypeStruct((G, H), jnp.float32),
                     jax.ShapeDtypeStruct((G, H), jnp.float32)]
    else:
        out_shape = [jax.ShapeDtypeStruct((N, H), jnp.float32),
                     jax.ShapeDtypeStruct((G, 2 * H), jnp.float32)]
    return pl.pallas_call(
        functools.partial(_post_body, is_last),
        out_shape=out_shape,
        in_specs=in_specs,
        scratch_shapes=[pltpu.VMEM((NP, H), jnp.float32),
                        pltpu.VMEM((G, H), jnp.float32)],
        interpret=interpret,
    )


_post_mid = _mk_post(False)
_post_last = _mk_post(True)


def kernel(x, edge_index, batch, conv1_W, conv1_b, conv2_W, conv2_b,
           conv3_W, conv3_b, n1_w, n1_b, n2_w, n2_b, n3_w, n3_b,
           lin1_W, lin1_b, lin2_W, lin2_b, lin3_W, lin3_b):
    r = edge_index[0]
    c = edge_index[1]
    rp = jnp.concatenate(
        [r, jnp.zeros((EPAD - E,), jnp.int32)])
    cp = jnp.concatenate(
        [c, jnp.full((EPAD - E,), NA - 1, jnp.int32)])
    r3 = rp.reshape(NW, NCE, KE)
    c3 = cp.reshape(NW, NCE, KE)
    batch2 = batch.astype(jnp.int32).reshape(N, 1)
    starts = jnp.searchsorted(
        batch, jnp.arange(G + 1, dtype=batch.dtype)).astype(jnp.int32)

    degp = _sc_deg(c)

    g1 = _tc_g1(x, conv1_W, degp)
    acc1 = _sc_edge(r3, c3, g1)

    xr0 = jnp.zeros((G, 2 * H), jnp.float32)
    b1 = conv1_b.reshape(1, H)
    dummy_b = jnp.zeros((1, H), jnp.float32)
    dummy_w = jnp.zeros((H, H), jnp.float32)
    g2, xr1 = _post_mid(degp, acc1, g1, b1, n1_w.reshape(1, H),
                        n1_b.reshape(1, H), batch2, starts, xr0,
                        conv2_W, dummy_w, dummy_w, dummy_b, dummy_b, dummy_b)
    acc2 = _sc_edge(r3, c3, g2)

    g3, xr2 = _post_mid(degp, acc2, g2, conv2_b.reshape(1, H),
                        n2_w.reshape(1, H), n2_b.reshape(1, H), batch2,
                        starts, xr1, conv3_W, dummy_w, dummy_w,
                        dummy_b, dummy_b, dummy_b)
    acc3 = _sc_edge(r3, c3, g3)

    lin3p = jnp.zeros((H // 2, H), jnp.float32).at[:, :C].set(lin3_W)
    lb3p = jnp.zeros((1, H), jnp.float32).at[:, :C].set(lin3_b.reshape(1, C))
    lb2p = lin2_b.reshape(1, H // 2)
    outp, _ = _post_last(degp, acc3, g3, conv3_b.reshape(1, H),
                         n3_w.reshape(1, H), n3_b.reshape(1, H), batch2,
                         starts, xr2, lin1_W, lin2_W, lin3p,
                         lin1_b.reshape(1, H), lb2p, lb3p)
    return outp[:, :C]


# speedup vs baseline: 2.1010x; 1.0211x over previous
import functools
import jax
import jax.numpy as jnp
from jax import lax
from jax.experimental import pallas as pl
from jax.experimental.pallas import tpu as pltpu
from jax.experimental.pallas import tpu_sc as plsc

N = 10000
E = 320000
H = 128
G = 64
C = 10
EPS = 1e-5

NC = 2
NS = 16
NW = NC * NS
EPW = E // NW
KE = 80
NCE = 125
NPAIR = (NCE - 1) // 2
EPAD = NW * NCE * KE
KD = 2000
NCD = EPW // KD
NA = 10240
TR = NA // NS
NP = 10752
CH = 512


def _sc_mesh():
    return plsc.VectorSubcoreMesh(
        core_axis_name="c", subcore_axis_name="s",
        num_cores=NC, num_subcores=NS)


def _sc_deg_body(c_hbm, out_hbm, cidx, ones_v, acc):
    core = lax.axis_index("c")
    sid = lax.axis_index("s")
    w = core * NS + sid

    def zrow(i, _):
        ones_v[i, :] = jnp.zeros((16,), jnp.float32)
        return 0
    lax.fori_loop(0, TR, zrow, 0)
    pltpu.sync_copy(ones_v.at[pl.ds(0, TR)], acc.at[pl.ds(sid * TR, TR)])

    def orow(i, _):
        ones_v[i, :] = jnp.ones((16,), jnp.float32)
        return 0
    lax.fori_loop(0, KD, orow, 0)
    plsc.subcore_barrier()

    base = w * EPW

    def body(j, _):
        pltpu.sync_copy(c_hbm.at[pl.ds(base + j * KD, KD)], cidx)
        pltpu.sync_copy(ones_v, acc.at[cidx], add=True)
        return 0
    lax.fori_loop(0, NCD, body, 0)
    plsc.subcore_barrier()
    pltpu.sync_copy(acc.at[pl.ds(sid * TR, TR)],
                    out_hbm.at[core, pl.ds(sid * TR, TR)])


def _sc_edge_body(r_hbm, c_hbm, g_hbm, out_hbm, ridx, cidx,
                  rows0, rows1, acc, sem0, sem1):
    core = lax.axis_index("c")
    sid = lax.axis_index("s")
    w = core * NS + sid

    def zrow(i, _):
        for j in range(H // 16):
            rows0[i, pl.ds(j * 16, 16)] = jnp.zeros((16,), jnp.float32)
        return 0
    lax.fori_loop(0, KE, zrow, 0)
    off = 0
    while off < TR:
        step = min(KE, TR - off)
        pltpu.sync_copy(rows0.at[pl.ds(0, step)],
                        acc.at[pl.ds(sid * TR + off, step)])
        off += step
    plsc.subcore_barrier()

    pltpu.sync_copy(r_hbm.at[w, pl.ds(0, NCE)], ridx)
    pltpu.sync_copy(c_hbm.at[w, pl.ds(0, NCE)], cidx)

    def gather(j, buf, sem):
        pltpu.async_copy(g_hbm.at[ridx.at[j]], buf, sem)

    def gwait(buf, sem):
        pltpu.make_async_copy(g_hbm.at[ridx.at[0]], buf, sem).wait()

    gather(0, rows0, sem0)

    def body(p, _):
        j0 = 2 * p
        gwait(rows0, sem0)
        gather(j0 + 1, rows1, sem1)
        pltpu.sync_copy(rows0, acc.at[cidx.at[j0]], add=True)
        gwait(rows1, sem1)
        gather(j0 + 2, rows0, sem0)
        pltpu.sync_copy(rows1, acc.at[cidx.at[j0 + 1]], add=True)
        return 0
    lax.fori_loop(0, NPAIR, body, 0)
    gwait(rows0, sem0)
    pltpu.sync_copy(rows0, acc.at[cidx.at[NCE - 1]], add=True)
    plsc.subcore_barrier()
    pltpu.sync_copy(acc.at[pl.ds(sid * TR, TR)],
                    out_hbm.at[core, pl.ds(sid * TR, TR)])


@functools.cache
def _get_sc_deg():
    return pl.kernel(
        _sc_deg_body,
        out_type=jax.ShapeDtypeStruct((NC, NA, 16), jnp.float32),
        mesh=_sc_mesh(),
        compiler_params=pltpu.CompilerParams(use_tc_tiling_on_sc=False),
        scratch_types=[
            pltpu.VMEM((KD,), jnp.int32),
            pltpu.VMEM((KD, 16), jnp.float32),
            pltpu.VMEM_SHARED((NA, 16), jnp.float32),
        ],
    )


@functools.cache
def _get_sc_edge():
    return pl.kernel(
        _sc_edge_body,
        out_type=jax.ShapeDtypeStruct((NC, NA, H), jnp.float32),
        mesh=_sc_mesh(),
        compiler_params=pltpu.CompilerParams(use_tc_tiling_on_sc=False),
        scratch_types=[
            pltpu.VMEM((NCE, KE), jnp.int32),
            pltpu.VMEM((NCE, KE), jnp.int32),
            pltpu.VMEM((KE, H), jnp.float32),
            pltpu.VMEM((KE, H), jnp.float32),
            pltpu.VMEM_SHARED((NA, H), jnp.float32),
            pltpu.SemaphoreType.DMA,
            pltpu.SemaphoreType.DMA,
        ],
    )


def _sc_deg(c):
    return _get_sc_deg()(c)


def _sc_edge(r3, c3, g):
    return _get_sc_edge()(r3, c3, g)


_HI = lax.Precision.HIGHEST


def _dinv_from(dp_ref):
    d0 = dp_ref[0]
    d1 = dp_ref[1]
    return lax.rsqrt(1.0 + d0[:N, 0:1] + d1[:N, 0:1])


def _tc_g1_body(x_ref, w_ref, dp_ref, g_ref):
    dinv = _dinv_from(dp_ref)
    g_ref[...] = dinv * lax.dot(x_ref[...], w_ref[...], precision=_HI)


def _mk_g1(interpret=False):
    return pl.pallas_call(
        _tc_g1_body,
        out_shape=jax.ShapeDtypeStruct((N, H), jnp.float32),
        interpret=interpret,
    )


_tc_g1 = _mk_g1()


def _leaky(v):
    return jnp.where(v > 0, v, 0.01 * v)


RB = 1000
NRB = N // RB


def _ln_stats(dp_ref, accp_ref, g_ref, b_ref, batch_ref, hs_ref, blk_dinv, blk_oh):
    def blk1(i, carry):
        s1, s2, cnt = carry
        r0 = pl.multiple_of(i * RB, 8)
        a = accp_ref[0, pl.ds(r0, RB), :] + accp_ref[1, pl.ds(r0, RB), :]
        out = blk_dinv(i) * (a + g_ref[pl.ds(r0, RB), :]) + b_ref[...]
        hs_ref[pl.ds(r0, RB), :] = out
        oh = blk_oh(i)
        dn = (((0,), (0,)), ((), ()))
        s1 = s1 + lax.dot_general(oh, out, dn, precision=_HI)
        s2 = s2 + lax.dot_general(oh, out * out, dn, precision=_HI)
        cnt = cnt + lax.dot_general(oh, jnp.ones((RB, 1), jnp.float32), dn,
                                    precision=_HI)
        return s1, s2, cnt

    z = jnp.zeros((G, H), jnp.float32)
    s1, s2, cnt = lax.fori_loop(
        0, NRB, blk1, (z, z, jnp.zeros((G, 1), jnp.float32)))
    cntf = jnp.maximum(cnt * H, 1.0)
    mean = jnp.sum(s1, axis=1, keepdims=True) / cntf
    sq = jnp.sum(s2, axis=1, keepdims=True) / cntf
    var = jnp.maximum(sq - mean * mean, 0.0)
    rsig = lax.rsqrt(var + EPS)
    return jnp.concatenate([mean, rsig], axis=1), cnt


def _mk_blk_helpers(dp_ref, batch_ref):
    def blk_dinv(i):
        r0 = pl.multiple_of(i * RB, 8)
        d0 = dp_ref[0, pl.ds(r0, RB), :]
        d1 = dp_ref[1, pl.ds(r0, RB), :]
        return lax.rsqrt(1.0 + d0[:, 0:1] + d1[:, 0:1])

    def blk_oh(i):
        r0 = pl.multiple_of(i * RB, 8)
        bb = batch_ref[pl.ds(r0, RB), :]
        return (bb == lax.broadcasted_iota(jnp.int32, (RB, G), 1)
                ).astype(jnp.float32)
    return blk_dinv, blk_oh


def _gmp_from(hs_ref, gmp_ref, starts_ref):
    def graph_body(gi, _):
        start = starts_ref[gi]
        num = starts_ref[gi + 1] - start
        nch = (num + CH - 1) // CH

        def chunk_body(kk, m):
            astart = ((start + kk * CH) // 8) * 8
            astart = pl.multiple_of(astart, 8)
            win = hs_ref[pl.ds(astart, CH + 8), :]
            ridx = astart + lax.broadcasted_iota(jnp.int32, (CH + 8, 1), 0)
            lo = start + kk * CH
            hi = start + jnp.minimum((kk + 1) * CH, num)
            valid = (ridx >= lo) & (ridx < hi)
            vals = jnp.where(valid, win, -jnp.inf)
            return jnp.maximum(m, jnp.max(vals, axis=0, keepdims=True))

        m = lax.fori_loop(0, nch, chunk_body, jnp.full((1, H), -jnp.inf))
        m = jnp.where(num > 0, m, jnp.zeros((1, H), jnp.float32))
        gmp_ref[pl.ds(gi, 1), :] = m
        return 0
    lax.fori_loop(0, G, graph_body, 0)


def _post_mid_body(dp_ref, accp_ref, g_ref, b_ref, nw_ref, nb_ref,
                   batch_ref, wa_ref, o0_ref, o1_ref, hs_ref):
    blk_dinv, blk_oh = _mk_blk_helpers(dp_ref, batch_ref)
    stats, _ = _ln_stats(dp_ref, accp_ref, g_ref, b_ref, batch_ref, hs_ref,
                         blk_dinv, blk_oh)

    def blk2(i, _):
        r0 = pl.multiple_of(i * RB, 8)
        out = hs_ref[pl.ds(r0, RB), :]
        oh = blk_oh(i)
        nst = lax.dot_general(oh, stats, (((1,), (0,)), ((), ())),
                              precision=_HI)
        xn = (out - nst[:, 0:1]) * nst[:, 1:2] * nw_ref[...] + nb_ref[...]
        h = _leaky(xn)
        o1_ref[pl.ds(r0, RB), :] = h
        o0_ref[pl.ds(r0, RB), :] = blk_dinv(i) * lax.dot(
            h, wa_ref[...], precision=_HI)
        return 0
    lax.fori_loop(0, NRB, blk2, 0)


def _pool_body(h_ref, batch_ref, starts_ref, xr_ref, o_ref, hs_ref, gmp_ref):
    _, blk_oh = _mk_blk_helpers(None, batch_ref)

    def blk(i, carry):
        gs, cnt = carry
        r0 = pl.multiple_of(i * RB, 8)
        h = h_ref[pl.ds(r0, RB), :]
        hs_ref[pl.ds(r0, RB), :] = h
        oh = blk_oh(i)
        dn = (((0,), (0,)), ((), ()))
        gs = gs + lax.dot_general(oh, h, dn, precision=_HI)
        cnt = cnt + lax.dot_general(oh, jnp.ones((RB, 1), jnp.float32), dn,
                                    precision=_HI)
        return gs, cnt

    gs, cnt = lax.fori_loop(
        0, NRB, blk,
        (jnp.zeros((G, H), jnp.float32), jnp.zeros((G, 1), jnp.float32)))
    gap = gs / jnp.maximum(cnt, 1.0)
    _gmp_from(hs_ref, gmp_ref, starts_ref)
    o_ref[...] = jnp.concatenate([gmp_ref[...], gap], axis=1) + xr_ref[...]


def _post_last_body(dp_ref, accp_ref, g_ref, b_ref, nw_ref, nb_ref,
                    batch_ref, starts_ref, xr_ref, wa_ref, wb_ref, wc_ref,
                    ba_ref, bb_ref, bc_ref, o0_ref, hs_ref, gmp_ref):
    blk_dinv, blk_oh = _mk_blk_helpers(dp_ref, batch_ref)
    stats, cnt = _ln_stats(dp_ref, accp_ref, g_ref, b_ref, batch_ref, hs_ref,
                           blk_dinv, blk_oh)

    def blk2(i, gs):
        r0 = pl.multiple_of(i * RB, 8)
        out = hs_ref[pl.ds(r0, RB), :]
        oh = blk_oh(i)
        nst = lax.dot_general(oh, stats, (((1,), (0,)), ((), ())),
                              precision=_HI)
        xn = (out - nst[:, 0:1]) * nst[:, 1:2] * nw_ref[...] + nb_ref[...]
        h = _leaky(xn)
        hs_ref[pl.ds(r0, RB), :] = h
        return gs + lax.dot_general(oh, h, (((0,), (0,)), ((), ())),
                                    precision=_HI)

    gs = lax.fori_loop(0, NRB, blk2, jnp.zeros((G, H), jnp.float32))
    gap = gs / jnp.maximum(cnt, 1.0)
    _gmp_from(hs_ref, gmp_ref, starts_ref)
    xp = jnp.concatenate([gmp_ref[...], gap], axis=1) + xr_ref[...]

    z1 = _leaky(lax.dot(xp, wa_ref[...], precision=_HI) + ba_ref[...])
    z2 = _leaky(lax.dot(z1, wb_ref[...], precision=_HI) + bb_ref[...])
    logits = lax.dot(z2, wc_ref[...], precision=_HI) + bc_ref[...]
    colid = lax.broadcasted_iota(jnp.int32, (G, H), 1)
    lmask = colid < C
    lw = jnp.where(lmask, logits, -jnp.inf)
    mx = jnp.max(lw, axis=1, keepdims=True)
    ex = jnp.where(lmask, jnp.exp(lw - mx), 0.0)
    lse = jnp.log(jnp.sum(ex, axis=1, keepdims=True)) + mx
    o0_ref[...] = logits - lse


def _mk_post_mid(interpret=False):
    return pl.pallas_call(
        _post_mid_body,
        out_shape=[jax.ShapeDtypeStruct((N, H), jnp.float32),
                   jax.ShapeDtypeStruct((N, H), jnp.float32)],
        in_specs=[pl.BlockSpec(memory_space=pltpu.VMEM) for _ in range(8)],
        scratch_shapes=[pltpu.VMEM((NP, H), jnp.float32)],
        interpret=interpret,
    )


def _mk_pool(interpret=False):
    in_specs = [pl.BlockSpec(memory_space=pltpu.VMEM) for _ in range(4)]
    in_specs[2] = pl.BlockSpec(memory_space=pltpu.SMEM)
    return pl.pallas_call(
        _pool_body,
        out_shape=jax.ShapeDtypeStruct((G, 2 * H), jnp.float32),
        in_specs=in_specs,
        scratch_shapes=[pltpu.VMEM((NP, H), jnp.float32),
                        pltpu.VMEM((G, H), jnp.float32)],
        interpret=interpret,
    )


def _mk_post_last(interpret=False):
    in_specs = [pl.BlockSpec(memory_space=pltpu.VMEM) for _ in range(15)]
    in_specs[7] = pl.BlockSpec(memory_space=pltpu.SMEM)
    return pl.pallas_call(
        _post_last_body,
        out_shape=jax.ShapeDtypeStruct((G, H), jnp.float32),
        in_specs=in_specs,
        scratch_shapes=[pltpu.VMEM((NP, H), jnp.float32),
                        pltpu.VMEM((G, H), jnp.float32)],
        interpret=interpret,
    )


_post_mid = _mk_post_mid()
_pool = _mk_pool()
_post_last = _mk_post_last()


def kernel(x, edge_index, batch, conv1_W, conv1_b, conv2_W, conv2_b,
           conv3_W, conv3_b, n1_w, n1_b, n2_w, n2_b, n3_w, n3_b,
           lin1_W, lin1_b, lin2_W, lin2_b, lin3_W, lin3_b):
    r = edge_index[0]
    c = edge_index[1]
    rp = jnp.concatenate(
        [r, jnp.zeros((EPAD - E,), jnp.int32)])
    cp = jnp.concatenate(
        [c, jnp.full((EPAD - E,), NA - 1, jnp.int32)])
    r3 = rp.reshape(NW, NCE, KE)
    c3 = cp.reshape(NW, NCE, KE)
    batch2 = batch.astype(jnp.int32).reshape(N, 1)
    starts = jnp.searchsorted(
        batch, jnp.arange(G + 1, dtype=batch.dtype)).astype(jnp.int32)

    degp = _sc_deg(c)

    g1 = _tc_g1(x, conv1_W, degp)
    acc1 = _sc_edge(r3, c3, g1)

    xr0 = jnp.zeros((G, 2 * H), jnp.float32)
    g2, h1 = _post_mid(degp, acc1, g1, conv1_b.reshape(1, H),
                       n1_w.reshape(1, H), n1_b.reshape(1, H), batch2,
                       conv2_W)
    acc2 = _sc_edge(r3, c3, g2)
    xr1 = _pool(h1, batch2, starts, xr0)

    g3, h2 = _post_mid(degp, acc2, g2, conv2_b.reshape(1, H),
                       n2_w.reshape(1, H), n2_b.reshape(1, H), batch2,
                       conv3_W)
    acc3 = _sc_edge(r3, c3, g3)
    xr2 = _pool(h2, batch2, starts, xr1)

    lin3p = jnp.zeros((H // 2, H), jnp.float32).at[:, :C].set(lin3_W)
    lb3p = jnp.zeros((1, H), jnp.float32).at[:, :C].set(lin3_b.reshape(1, C))
    lb2p = lin2_b.reshape(1, H // 2)
    outp = _post_last(degp, acc3, g3, conv3_b.reshape(1, H),
                      n3_w.reshape(1, H), n3_b.reshape(1, H), batch2,
                      starts, xr2, lin1_W, lin2_W, lin3p,
                      lin1_b.reshape(1, H), lb2p, lb3p)
    return outp[:, :C]


# speedup vs baseline: 2.3190x; 1.1037x over previous
import functools
import jax
import jax.numpy as jnp
from jax import lax
from jax.experimental import pallas as pl
from jax.experimental.pallas import tpu as pltpu
from jax.experimental.pallas import tpu_sc as plsc

N = 10000
E = 320000
H = 128
G = 64
C = 10
EPS = 1e-5

NC = 2
NS = 16
NW = NC * NS
EPW = E // NW
KE = 80
NCE = 125
NPAIR = (NCE - 1) // 2
EPAD = NW * NCE * KE
KD = 2000
NCD = EPW // KD
NA = 10240
TR = NA // NS
NP = 10752
CH = 512


def _sc_mesh():
    return plsc.VectorSubcoreMesh(
        core_axis_name="c", subcore_axis_name="s",
        num_cores=NC, num_subcores=NS)


def _sc_deg_body(c_hbm, out_hbm, cidx, ones_v, acc):
    core = lax.axis_index("c")
    sid = lax.axis_index("s")
    w = core * NS + sid

    def zrow(i, _):
        ones_v[i, :] = jnp.zeros((16,), jnp.float32)
        return 0
    lax.fori_loop(0, TR, zrow, 0)
    pltpu.sync_copy(ones_v.at[pl.ds(0, TR)], acc.at[pl.ds(sid * TR, TR)])

    def orow(i, _):
        ones_v[i, :] = jnp.ones((16,), jnp.float32)
        return 0
    lax.fori_loop(0, KD, orow, 0)
    plsc.subcore_barrier()

    base = w * EPW

    def body(j, _):
        pltpu.sync_copy(c_hbm.at[pl.ds(base + j * KD, KD)], cidx)
        pltpu.sync_copy(ones_v, acc.at[cidx], add=True)
        return 0
    lax.fori_loop(0, NCD, body, 0)
    plsc.subcore_barrier()
    pltpu.sync_copy(acc.at[pl.ds(sid * TR, TR)],
                    out_hbm.at[core, pl.ds(sid * TR, TR)])


def _sc_edge_body(r_hbm, c_hbm, g_hbm, out_hbm, ridx, cidx,
                  rows0, rows1, acc, sem0, sem1):
    core = lax.axis_index("c")
    sid = lax.axis_index("s")
    w = core * NS + sid

    def zrow(i, _):
        for j in range(H // 16):
            rows0[i, pl.ds(j * 16, 16)] = jnp.zeros((16,), jnp.float32)
        return 0
    lax.fori_loop(0, KE, zrow, 0)
    off = 0
    while off < TR:
        step = min(KE, TR - off)
        pltpu.sync_copy(rows0.at[pl.ds(0, step)],
                        acc.at[pl.ds(sid * TR + off, step)])
        off += step
    plsc.subcore_barrier()

    pltpu.sync_copy(r_hbm.at[w, pl.ds(0, NCE)], ridx)
    pltpu.sync_copy(c_hbm.at[w, pl.ds(0, NCE)], cidx)

    def gather(j, buf, sem):
        pltpu.async_copy(g_hbm.at[ridx.at[j]], buf, sem)

    def gwait(buf, sem):
        pltpu.make_async_copy(g_hbm.at[ridx.at[0]], buf, sem).wait()

    gather(0, rows0, sem0)

    def body(p, _):
        j0 = 2 * p
        gwait(rows0, sem0)
        gather(j0 + 1, rows1, sem1)
        pltpu.sync_copy(rows0, acc.at[cidx.at[j0]], add=True)
        gwait(rows1, sem1)
        gather(j0 + 2, rows0, sem0)
        pltpu.sync_copy(rows1, acc.at[cidx.at[j0 + 1]], add=True)
        return 0
    lax.fori_loop(0, NPAIR, body, 0)
    gwait(rows0, sem0)
    pltpu.sync_copy(rows0, acc.at[cidx.at[NCE - 1]], add=True)
    plsc.subcore_barrier()
    pltpu.sync_copy(acc.at[pl.ds(sid * TR, TR)],
                    out_hbm.at[core, pl.ds(sid * TR, TR)])


@functools.cache
def _get_sc_deg():
    return pl.kernel(
        _sc_deg_body,
        out_type=jax.ShapeDtypeStruct((NC, NA, 16), jnp.float32),
        mesh=_sc_mesh(),
        compiler_params=pltpu.CompilerParams(use_tc_tiling_on_sc=False),
        scratch_types=[
            pltpu.VMEM((KD,), jnp.int32),
            pltpu.VMEM((KD, 16), jnp.float32),
            pltpu.VMEM_SHARED((NA, 16), jnp.float32),
        ],
    )


@functools.cache
def _get_sc_edge():
    return pl.kernel(
        _sc_edge_body,
        out_type=jax.ShapeDtypeStruct((NC, NA, H), jnp.float32),
        mesh=_sc_mesh(),
        compiler_params=pltpu.CompilerParams(use_tc_tiling_on_sc=False),
        scratch_types=[
            pltpu.VMEM((NCE, KE), jnp.int32),
            pltpu.VMEM((NCE, KE), jnp.int32),
            pltpu.VMEM((KE, H), jnp.float32),
            pltpu.VMEM((KE, H), jnp.float32),
            pltpu.VMEM_SHARED((NA, H), jnp.float32),
            pltpu.SemaphoreType.DMA,
            pltpu.SemaphoreType.DMA,
        ],
    )


def _sc_deg(c):
    return _get_sc_deg()(c)


def _sc_edge(r3, c3, g):
    return _get_sc_edge()(r3, c3, g)


_HI = lax.Precision.HIGHEST
_LO = lax.Precision.DEFAULT


def _dinv_from(dp_ref):
    d0 = dp_ref[0]
    d1 = dp_ref[1]
    return lax.rsqrt(1.0 + d0[:N, 0:1] + d1[:N, 0:1])


def _tc_g1_body(x_ref, w_ref, dp_ref, g_ref):
    dinv = _dinv_from(dp_ref)
    g_ref[...] = dinv * lax.dot(x_ref[...], w_ref[...], precision=_HI)


def _mk_g1(interpret=False):
    return pl.pallas_call(
        _tc_g1_body,
        out_shape=jax.ShapeDtypeStruct((N, H), jnp.float32),
        interpret=interpret,
    )


_tc_g1 = _mk_g1()


def _leaky(v):
    return jnp.where(v > 0, v, 0.01 * v)


RB = 1000
NRB = N // RB


def _ln_stats(dp_ref, accp_ref, g_ref, b_ref, batch_ref, hs_ref, blk_dinv, blk_oh):
    def blk1(i, carry):
        s1, s2, cnt = carry
        r0 = pl.multiple_of(i * RB, 8)
        a = accp_ref[0, pl.ds(r0, RB), :] + accp_ref[1, pl.ds(r0, RB), :]
        out = blk_dinv(i) * (a + g_ref[pl.ds(r0, RB), :]) + b_ref[...]
        hs_ref[pl.ds(r0, RB), :] = out
        oh = blk_oh(i)
        dn = (((0,), (0,)), ((), ()))
        s1 = s1 + lax.dot_general(oh, out, dn, precision=_LO)
        s2 = s2 + lax.dot_general(oh, out * out, dn, precision=_LO)
        cnt = cnt + lax.dot_general(oh, jnp.ones((RB, 1), jnp.float32), dn,
                                    precision=_LO)
        return s1, s2, cnt

    z = jnp.zeros((G, H), jnp.float32)
    s1, s2, cnt = lax.fori_loop(
        0, NRB, blk1, (z, z, jnp.zeros((G, 1), jnp.float32)))
    cntf = jnp.maximum(cnt * H, 1.0)
    mean = jnp.sum(s1, axis=1, keepdims=True) / cntf
    sq = jnp.sum(s2, axis=1, keepdims=True) / cntf
    var = jnp.maximum(sq - mean * mean, 0.0)
    rsig = lax.rsqrt(var + EPS)
    return jnp.concatenate([mean, rsig], axis=1), cnt


def _mk_blk_helpers(dp_ref, batch_ref):
    def blk_dinv(i):
        r0 = pl.multiple_of(i * RB, 8)
        d0 = dp_ref[0, pl.ds(r0, RB), :]
        d1 = dp_ref[1, pl.ds(r0, RB), :]
        return lax.rsqrt(1.0 + d0[:, 0:1] + d1[:, 0:1])

    def blk_oh(i):
        r0 = pl.multiple_of(i * RB, 8)
        bb = batch_ref[pl.ds(r0, RB), :]
        return (bb == lax.broadcasted_iota(jnp.int32, (RB, G), 1)
                ).astype(jnp.float32)
    return blk_dinv, blk_oh


def _gmp_from(hs_ref, gmp_ref, starts_ref):
    def graph_body(gi, _):
        start = starts_ref[gi]
        num = starts_ref[gi + 1] - start
        nch = (num + CH - 1) // CH

        def chunk_body(kk, m):
            astart = ((start + kk * CH) // 8) * 8
            astart = pl.multiple_of(astart, 8)
            win = hs_ref[pl.ds(astart, CH + 8), :]
            ridx = astart + lax.broadcasted_iota(jnp.int32, (CH + 8, 1), 0)
            lo = start + kk * CH
            hi = start + jnp.minimum((kk + 1) * CH, num)
            valid = (ridx >= lo) & (ridx < hi)
            vals = jnp.where(valid, win, -jnp.inf)
            return jnp.maximum(m, jnp.max(vals, axis=0, keepdims=True))

        m = lax.fori_loop(0, nch, chunk_body, jnp.full((1, H), -jnp.inf))
        m = jnp.where(num > 0, m, jnp.zeros((1, H), jnp.float32))
        gmp_ref[pl.ds(gi, 1), :] = m
        return 0
    lax.fori_loop(0, G, graph_body, 0)


def _post_mid_body(dp_ref, accp_ref, g_ref, b_ref, nw_ref, nb_ref,
                   batch_ref, wa_ref, o0_ref, o1_ref, hs_ref):
    blk_dinv, blk_oh = _mk_blk_helpers(dp_ref, batch_ref)
    stats, _ = _ln_stats(dp_ref, accp_ref, g_ref, b_ref, batch_ref, hs_ref,
                         blk_dinv, blk_oh)

    def blk2(i, _):
        r0 = pl.multiple_of(i * RB, 8)
        out = hs_ref[pl.ds(r0, RB), :]
        oh = blk_oh(i)
        nst = lax.dot_general(oh, stats, (((1,), (0,)), ((), ())),
                              precision=_LO)
        xn = (out - nst[:, 0:1]) * nst[:, 1:2] * nw_ref[...] + nb_ref[...]
        h = _leaky(xn)
        o1_ref[pl.ds(r0, RB), :] = h
        o0_ref[pl.ds(r0, RB), :] = blk_dinv(i) * lax.dot(
            h, wa_ref[...], precision=_HI)
        return 0
    lax.fori_loop(0, NRB, blk2, 0)


def _pool_body(h_ref, batch_ref, starts_ref, xr_ref, o_ref, hs_ref, gmp_ref):
    _, blk_oh = _mk_blk_helpers(None, batch_ref)

    def blk(i, carry):
        gs, cnt = carry
        r0 = pl.multiple_of(i * RB, 8)
        h = h_ref[pl.ds(r0, RB), :]
        hs_ref[pl.ds(r0, RB), :] = h
        oh = blk_oh(i)
        dn = (((0,), (0,)), ((), ()))
        gs = gs + lax.dot_general(oh, h, dn, precision=_LO)
        cnt = cnt + lax.dot_general(oh, jnp.ones((RB, 1), jnp.float32), dn,
                                    precision=_LO)
        return gs, cnt

    gs, cnt = lax.fori_loop(
        0, NRB, blk,
        (jnp.zeros((G, H), jnp.float32), jnp.zeros((G, 1), jnp.float32)))
    gap = gs / jnp.maximum(cnt, 1.0)
    _gmp_from(hs_ref, gmp_ref, starts_ref)
    o_ref[...] = jnp.concatenate([gmp_ref[...], gap], axis=1) + xr_ref[...]


def _post_last_body(dp_ref, accp_ref, g_ref, b_ref, nw_ref, nb_ref,
                    batch_ref, starts_ref, xr_ref, wa_ref, wb_ref, wc_ref,
                    ba_ref, bb_ref, bc_ref, o0_ref, hs_ref, gmp_ref):
    blk_dinv, blk_oh = _mk_blk_helpers(dp_ref, batch_ref)
    stats, cnt = _ln_stats(dp_ref, accp_ref, g_ref, b_ref, batch_ref, hs_ref,
                           blk_dinv, blk_oh)

    def blk2(i, gs):
        r0 = pl.multiple_of(i * RB, 8)
        out = hs_ref[pl.ds(r0, RB), :]
        oh = blk_oh(i)
        nst = lax.dot_general(oh, stats, (((1,), (0,)), ((), ())),
                              precision=_LO)
        xn = (out - nst[:, 0:1]) * nst[:, 1:2] * nw_ref[...] + nb_ref[...]
        h = _leaky(xn)
        hs_ref[pl.ds(r0, RB), :] = h
        return gs + lax.dot_general(oh, h, (((0,), (0,)), ((), ())),
                                    precision=_LO)

    gs = lax.fori_loop(0, NRB, blk2, jnp.zeros((G, H), jnp.float32))
    gap = gs / jnp.maximum(cnt, 1.0)
    _gmp_from(hs_ref, gmp_ref, starts_ref)
    xp = jnp.concatenate([gmp_ref[...], gap], axis=1) + xr_ref[...]

    z1 = _leaky(lax.dot(xp, wa_ref[...], precision=_HI) + ba_ref[...])
    z2 = _leaky(lax.dot(z1, wb_ref[...], precision=_HI) + bb_ref[...])
    logits = lax.dot(z2, wc_ref[...], precision=_HI) + bc_ref[...]
    colid = lax.broadcasted_iota(jnp.int32, (G, H), 1)
    lmask = colid < C
    lw = jnp.where(lmask, logits, -jnp.inf)
    mx = jnp.max(lw, axis=1, keepdims=True)
    ex = jnp.where(lmask, jnp.exp(lw - mx), 0.0)
    lse = jnp.log(jnp.sum(ex, axis=1, keepdims=True)) + mx
    o0_ref[...] = logits - lse


def _mk_post_mid(interpret=False):
    return pl.pallas_call(
        _post_mid_body,
        out_shape=[jax.ShapeDtypeStruct((N, H), jnp.float32),
                   jax.ShapeDtypeStruct((N, H), jnp.float32)],
        in_specs=[pl.BlockSpec(memory_space=pltpu.VMEM) for _ in range(8)],
        scratch_shapes=[pltpu.VMEM((NP, H), jnp.float32)],
        interpret=interpret,
    )


def _mk_pool(interpret=False):
    in_specs = [pl.BlockSpec(memory_space=pltpu.VMEM) for _ in range(4)]
    in_specs[2] = pl.BlockSpec(memory_space=pltpu.SMEM)
    return pl.pallas_call(
        _pool_body,
        out_shape=jax.ShapeDtypeStruct((G, 2 * H), jnp.float32),
        in_specs=in_specs,
        scratch_shapes=[pltpu.VMEM((NP, H), jnp.float32),
                        pltpu.VMEM((G, H), jnp.float32)],
        interpret=interpret,
    )


def _mk_post_last(interpret=False):
    in_specs = [pl.BlockSpec(memory_space=pltpu.VMEM) for _ in range(15)]
    in_specs[7] = pl.BlockSpec(memory_space=pltpu.SMEM)
    return pl.pallas_call(
        _post_last_body,
        out_shape=jax.ShapeDtypeStruct((G, H), jnp.float32),
        in_specs=in_specs,
        scratch_shapes=[pltpu.VMEM((NP, H), jnp.float32),
                        pltpu.VMEM((G, H), jnp.float32)],
        interpret=interpret,
    )


_post_mid = _mk_post_mid()
_pool = _mk_pool()
_post_last = _mk_post_last()


def kernel(x, edge_index, batch, conv1_W, conv1_b, conv2_W, conv2_b,
           conv3_W, conv3_b, n1_w, n1_b, n2_w, n2_b, n3_w, n3_b,
           lin1_W, lin1_b, lin2_W, lin2_b, lin3_W, lin3_b):
    r = edge_index[0]
    c = edge_index[1]
    rp = jnp.concatenate(
        [r, jnp.zeros((EPAD - E,), jnp.int32)])
    cp = jnp.concatenate(
        [c, jnp.full((EPAD - E,), NA - 1, jnp.int32)])
    r3 = rp.reshape(NW, NCE, KE)
    c3 = cp.reshape(NW, NCE, KE)
    batch2 = batch.astype(jnp.int32).reshape(N, 1)
    starts = jnp.searchsorted(
        batch, jnp.arange(G + 1, dtype=batch.dtype)).astype(jnp.int32)

    degp = _sc_deg(c)

    g1 = _tc_g1(x, conv1_W, degp)
    acc1 = _sc_edge(r3, c3, g1)

    xr0 = jnp.zeros((G, 2 * H), jnp.float32)
    g2, h1 = _post_mid(degp, acc1, g1, conv1_b.reshape(1, H),
                       n1_w.reshape(1, H), n1_b.reshape(1, H), batch2,
                       conv2_W)
    acc2 = _sc_edge(r3, c3, g2)
    xr1 = _pool(h1, batch2, starts, xr0)

    g3, h2 = _post_mid(degp, acc2, g2, conv2_b.reshape(1, H),
                       n2_w.reshape(1, H), n2_b.reshape(1, H), batch2,
                       conv3_W)
    acc3 = _sc_edge(r3, c3, g3)
    xr2 = _pool(h2, batch2, starts, xr1)

    lin3p = jnp.zeros((H // 2, H), jnp.float32).at[:, :C].set(lin3_W)
    lb3p = jnp.zeros((1, H), jnp.float32).at[:, :C].set(lin3_b.reshape(1, C))
    lb2p = lin2_b.reshape(1, H // 2)
    outp = _post_last(degp, acc3, g3, conv3_b.reshape(1, H),
                      n3_w.reshape(1, H), n3_b.reshape(1, H), batch2,
                      starts, xr2, lin1_W, lin2_W, lin3p,
                      lin1_b.reshape(1, H), lb2p, lb3p)
    return outp[:, :C]


# speedup vs baseline: 2.3911x; 1.0311x over previous
import functools
import jax
import jax.numpy as jnp
from jax import lax
from jax.experimental import pallas as pl
from jax.experimental.pallas import tpu as pltpu
from jax.experimental.pallas import tpu_sc as plsc

N = 10000
E = 320000
H = 128
G = 64
C = 10
EPS = 1e-5

NC = 2
NS = 16
NW = NC * NS
EPW = E // NW
KE = 80
NCE = 125
NPAIR = (NCE - 1) // 2
EPAD = NW * NCE * KE
KD = 2000
NCD = EPW // KD
NA = 10240
TR = NA // NS
NP = 10752
CH = 512


def _sc_mesh():
    return plsc.VectorSubcoreMesh(
        core_axis_name="c", subcore_axis_name="s",
        num_cores=NC, num_subcores=NS)


def _sc_deg_body(c_hbm, out_hbm, cidx, ones_v, acc):
    core = lax.axis_index("c")
    sid = lax.axis_index("s")
    w = core * NS + sid

    def zrow(i, _):
        ones_v[i, :] = jnp.zeros((16,), jnp.float32)
        return 0
    lax.fori_loop(0, TR, zrow, 0)
    pltpu.sync_copy(ones_v.at[pl.ds(0, TR)], acc.at[pl.ds(sid * TR, TR)])

    def orow(i, _):
        ones_v[i, :] = jnp.ones((16,), jnp.float32)
        return 0
    lax.fori_loop(0, KD, orow, 0)
    plsc.subcore_barrier()

    base = w * EPW

    def body(j, _):
        pltpu.sync_copy(c_hbm.at[pl.ds(base + j * KD, KD)], cidx)
        pltpu.sync_copy(ones_v, acc.at[cidx], add=True)
        return 0
    lax.fori_loop(0, NCD, body, 0)
    plsc.subcore_barrier()
    pltpu.sync_copy(acc.at[pl.ds(sid * TR, TR)],
                    out_hbm.at[core, pl.ds(sid * TR, TR)])


def _sc_edge_body(r_hbm, c_hbm, g_hbm, out_hbm, ridx, cidx,
                  rows0, rows1, acc, sem0, sem1):
    core = lax.axis_index("c")
    sid = lax.axis_index("s")
    w = core * NS + sid

    def zrow(i, _):
        for j in range(H // 16):
            rows0[i, pl.ds(j * 16, 16)] = jnp.zeros((16,), jnp.float32)
        return 0
    lax.fori_loop(0, KE, zrow, 0)
    off = 0
    while off < TR:
        step = min(KE, TR - off)
        pltpu.sync_copy(rows0.at[pl.ds(0, step)],
                        acc.at[pl.ds(sid * TR + off, step)])
        off += step
    plsc.subcore_barrier()

    pltpu.sync_copy(r_hbm.at[w, pl.ds(0, NCE)], ridx)
    pltpu.sync_copy(c_hbm.at[w, pl.ds(0, NCE)], cidx)

    def gather(j, buf, sem):
        pltpu.async_copy(g_hbm.at[ridx.at[j]], buf, sem)

    def gwait(buf, sem):
        pltpu.make_async_copy(g_hbm.at[ridx.at[0]], buf, sem).wait()

    gather(0, rows0, sem0)

    def body(p, _):
        j0 = 2 * p
        gwait(rows0, sem0)
        gather(j0 + 1, rows1, sem1)
        pltpu.sync_copy(rows0, acc.at[cidx.at[j0]], add=True)
        gwait(rows1, sem1)
        gather(j0 + 2, rows0, sem0)
        pltpu.sync_copy(rows1, acc.at[cidx.at[j0 + 1]], add=True)
        return 0
    lax.fori_loop(0, NPAIR, body, 0)
    gwait(rows0, sem0)
    pltpu.sync_copy(rows0, acc.at[cidx.at[NCE - 1]], add=True)
    plsc.subcore_barrier()
    pltpu.sync_copy(acc.at[pl.ds(sid * TR, TR)],
                    out_hbm.at[core, pl.ds(sid * TR, TR)])


@functools.cache
def _get_sc_deg():
    return pl.kernel(
        _sc_deg_body,
        out_type=jax.ShapeDtypeStruct((NC, NA, 16), jnp.float32),
        mesh=_sc_mesh(),
        compiler_params=pltpu.CompilerParams(use_tc_tiling_on_sc=False),
        scratch_types=[
            pltpu.VMEM((KD,), jnp.int32),
            pltpu.VMEM((KD, 16), jnp.float32),
            pltpu.VMEM_SHARED((NA, 16), jnp.float32),
        ],
    )


@functools.cache
def _get_sc_edge():
    return pl.kernel(
        _sc_edge_body,
        out_type=jax.ShapeDtypeStruct((NC, NA, H), jnp.float32),
        mesh=_sc_mesh(),
        compiler_params=pltpu.CompilerParams(use_tc_tiling_on_sc=False),
        scratch_types=[
            pltpu.VMEM((NCE, KE), jnp.int32),
            pltpu.VMEM((NCE, KE), jnp.int32),
            pltpu.VMEM((KE, H), jnp.float32),
            pltpu.VMEM((KE, H), jnp.float32),
            pltpu.VMEM_SHARED((NA, H), jnp.float32),
            pltpu.SemaphoreType.DMA,
            pltpu.SemaphoreType.DMA,
        ],
    )


def _sc_deg(c):
    return _get_sc_deg()(c)


def _sc_edge(r3, c3, g):
    return _get_sc_edge()(r3, c3, g)


_HI = lax.Precision.DEFAULT
_LO = lax.Precision.DEFAULT


def _dinv_from(dp_ref):
    d0 = dp_ref[0]
    d1 = dp_ref[1]
    return lax.rsqrt(1.0 + d0[:N, 0:1] + d1[:N, 0:1])


def _tc_g1_body(x_ref, w_ref, dp_ref, g_ref):
    dinv = _dinv_from(dp_ref)
    g_ref[...] = dinv * lax.dot(x_ref[...], w_ref[...], precision=_HI)


def _mk_g1(interpret=False):
    return pl.pallas_call(
        _tc_g1_body,
        out_shape=jax.ShapeDtypeStruct((N, H), jnp.float32),
        interpret=interpret,
    )


_tc_g1 = _mk_g1()


def _leaky(v):
    return jnp.where(v > 0, v, 0.01 * v)


RB = 1000
NRB = N // RB


def _ln_stats(dp_ref, accp_ref, g_ref, b_ref, batch_ref, hs_ref, blk_dinv, blk_oh):
    def blk1(i, carry):
        s1, s2, cnt = carry
        r0 = pl.multiple_of(i * RB, 8)
        a = accp_ref[0, pl.ds(r0, RB), :] + accp_ref[1, pl.ds(r0, RB), :]
        out = blk_dinv(i) * (a + g_ref[pl.ds(r0, RB), :]) + b_ref[...]
        hs_ref[pl.ds(r0, RB), :] = out
        oh = blk_oh(i)
        dn = (((0,), (0,)), ((), ()))
        s1 = s1 + lax.dot_general(oh, out, dn, precision=_LO)
        s2 = s2 + lax.dot_general(oh, out * out, dn, precision=_LO)
        cnt = cnt + lax.dot_general(oh, jnp.ones((RB, 1), jnp.float32), dn,
                                    precision=_LO)
        return s1, s2, cnt

    z = jnp.zeros((G, H), jnp.float32)
    s1, s2, cnt = lax.fori_loop(
        0, NRB, blk1, (z, z, jnp.zeros((G, 1), jnp.float32)))
    cntf = jnp.maximum(cnt * H, 1.0)
    mean = jnp.sum(s1, axis=1, keepdims=True) / cntf
    sq = jnp.sum(s2, axis=1, keepdims=True) / cntf
    var = jnp.maximum(sq - mean * mean, 0.0)
    rsig = lax.rsqrt(var + EPS)
    return jnp.concatenate([mean, rsig], axis=1), cnt


def _mk_blk_helpers(dp_ref, batch_ref):
    def blk_dinv(i):
        r0 = pl.multiple_of(i * RB, 8)
        d0 = dp_ref[0, pl.ds(r0, RB), :]
        d1 = dp_ref[1, pl.ds(r0, RB), :]
        return lax.rsqrt(1.0 + d0[:, 0:1] + d1[:, 0:1])

    def blk_oh(i):
        r0 = pl.multiple_of(i * RB, 8)
        bb = batch_ref[pl.ds(r0, RB), :]
        return (bb == lax.broadcasted_iota(jnp.int32, (RB, G), 1)
                ).astype(jnp.float32)
    return blk_dinv, blk_oh


def _gmp_from(hs_ref, gmp_ref, starts_ref):
    def graph_body(gi, _):
        start = starts_ref[gi]
        num = starts_ref[gi + 1] - start
        nch = (num + CH - 1) // CH

        def chunk_body(kk, m):
            astart = ((start + kk * CH) // 8) * 8
            astart = pl.multiple_of(astart, 8)
            win = hs_ref[pl.ds(astart, CH + 8), :]
            ridx = astart + lax.broadcasted_iota(jnp.int32, (CH + 8, 1), 0)
            lo = start + kk * CH
            hi = start + jnp.minimum((kk + 1) * CH, num)
            valid = (ridx >= lo) & (ridx < hi)
            vals = jnp.where(valid, win, -jnp.inf)
            return jnp.maximum(m, jnp.max(vals, axis=0, keepdims=True))

        m = lax.fori_loop(0, nch, chunk_body, jnp.full((1, H), -jnp.inf))
        m = jnp.where(num > 0, m, jnp.zeros((1, H), jnp.float32))
        gmp_ref[pl.ds(gi, 1), :] = m
        return 0
    lax.fori_loop(0, G, graph_body, 0)


def _post_mid_body(dp_ref, accp_ref, g_ref, b_ref, nw_ref, nb_ref,
                   batch_ref, wa_ref, o0_ref, o1_ref, hs_ref):
    blk_dinv, blk_oh = _mk_blk_helpers(dp_ref, batch_ref)
    stats, _ = _ln_stats(dp_ref, accp_ref, g_ref, b_ref, batch_ref, hs_ref,
                         blk_dinv, blk_oh)

    def blk2(i, _):
        r0 = pl.multiple_of(i * RB, 8)
        out = hs_ref[pl.ds(r0, RB), :]
        oh = blk_oh(i)
        nst = lax.dot_general(oh, stats, (((1,), (0,)), ((), ())),
                              precision=_LO)
        xn = (out - nst[:, 0:1]) * nst[:, 1:2] * nw_ref[...] + nb_ref[...]
        h = _leaky(xn)
        o1_ref[pl.ds(r0, RB), :] = h
        o0_ref[pl.ds(r0, RB), :] = blk_dinv(i) * lax.dot(
            h, wa_ref[...], precision=_HI)
        return 0
    lax.fori_loop(0, NRB, blk2, 0)


def _pool_body(h_ref, batch_ref, starts_ref, xr_ref, o_ref, hs_ref, gmp_ref):
    _, blk_oh = _mk_blk_helpers(None, batch_ref)

    def blk(i, carry):
        gs, cnt = carry
        r0 = pl.multiple_of(i * RB, 8)
        h = h_ref[pl.ds(r0, RB), :]
        hs_ref[pl.ds(r0, RB), :] = h
        oh = blk_oh(i)
        dn = (((0,), (0,)), ((), ()))
        gs = gs + lax.dot_general(oh, h, dn, precision=_LO)
        cnt = cnt + lax.dot_general(oh, jnp.ones((RB, 1), jnp.float32), dn,
                                    precision=_LO)
        return gs, cnt

    gs, cnt = lax.fori_loop(
        0, NRB, blk,
        (jnp.zeros((G, H), jnp.float32), jnp.zeros((G, 1), jnp.float32)))
    gap = gs / jnp.maximum(cnt, 1.0)
    _gmp_from(hs_ref, gmp_ref, starts_ref)
    o_ref[...] = jnp.concatenate([gmp_ref[...], gap], axis=1) + xr_ref[...]


def _post_last_body(dp_ref, accp_ref, g_ref, b_ref, nw_ref, nb_ref,
                    batch_ref, starts_ref, xr_ref, wa_ref, wb_ref, wc_ref,
                    ba_ref, bb_ref, bc_ref, o0_ref, hs_ref, gmp_ref):
    blk_dinv, blk_oh = _mk_blk_helpers(dp_ref, batch_ref)
    stats, cnt = _ln_stats(dp_ref, accp_ref, g_ref, b_ref, batch_ref, hs_ref,
                           blk_dinv, blk_oh)

    def blk2(i, gs):
        r0 = pl.multiple_of(i * RB, 8)
        out = hs_ref[pl.ds(r0, RB), :]
        oh = blk_oh(i)
        nst = lax.dot_general(oh, stats, (((1,), (0,)), ((), ())),
                              precision=_LO)
        xn = (out - nst[:, 0:1]) * nst[:, 1:2] * nw_ref[...] + nb_ref[...]
        h = _leaky(xn)
        hs_ref[pl.ds(r0, RB), :] = h
        return gs + lax.dot_general(oh, h, (((0,), (0,)), ((), ())),
                                    precision=_LO)

    gs = lax.fori_loop(0, NRB, blk2, jnp.zeros((G, H), jnp.float32))
    gap = gs / jnp.maximum(cnt, 1.0)
    _gmp_from(hs_ref, gmp_ref, starts_ref)
    xp = jnp.concatenate([gmp_ref[...], gap], axis=1) + xr_ref[...]

    z1 = _leaky(lax.dot(xp, wa_ref[...], precision=_HI) + ba_ref[...])
    z2 = _leaky(lax.dot(z1, wb_ref[...], precision=_HI) + bb_ref[...])
    logits = lax.dot(z2, wc_ref[...], precision=_HI) + bc_ref[...]
    colid = lax.broadcasted_iota(jnp.int32, (G, H), 1)
    lmask = colid < C
    lw = jnp.where(lmask, logits, -jnp.inf)
    mx = jnp.max(lw, axis=1, keepdims=True)
    ex = jnp.where(lmask, jnp.exp(lw - mx), 0.0)
    lse = jnp.log(jnp.sum(ex, axis=1, keepdims=True)) + mx
    o0_ref[...] = logits - lse


def _mk_post_mid(interpret=False):
    return pl.pallas_call(
        _post_mid_body,
        out_shape=[jax.ShapeDtypeStruct((N, H), jnp.float32),
                   jax.ShapeDtypeStruct((N, H), jnp.float32)],
        in_specs=[pl.BlockSpec(memory_space=pltpu.VMEM) for _ in range(8)],
        scratch_shapes=[pltpu.VMEM((NP, H), jnp.float32)],
        interpret=interpret,
    )


def _mk_pool(interpret=False):
    in_specs = [pl.BlockSpec(memory_space=pltpu.VMEM) for _ in range(4)]
    in_specs[2] = pl.BlockSpec(memory_space=pltpu.SMEM)
    return pl.pallas_call(
        _pool_body,
        out_shape=jax.ShapeDtypeStruct((G, 2 * H), jnp.float32),
        in_specs=in_specs,
        scratch_shapes=[pltpu.VMEM((NP, H), jnp.float32),
                        pltpu.VMEM((G, H), jnp.float32)],
        interpret=interpret,
    )


def _mk_post_last(interpret=False):
    in_specs = [pl.BlockSpec(memory_space=pltpu.VMEM) for _ in range(15)]
    in_specs[7] = pl.BlockSpec(memory_space=pltpu.SMEM)
    return pl.pallas_call(
        _post_last_body,
        out_shape=jax.ShapeDtypeStruct((G, H), jnp.float32),
        in_specs=in_specs,
        scratch_shapes=[pltpu.VMEM((NP, H), jnp.float32),
                        pltpu.VMEM((G, H), jnp.float32)],
        interpret=interpret,
    )


_post_mid = _mk_post_mid()
_pool = _mk_pool()
_post_last = _mk_post_last()


def kernel(x, edge_index, batch, conv1_W, conv1_b, conv2_W, conv2_b,
           conv3_W, conv3_b, n1_w, n1_b, n2_w, n2_b, n3_w, n3_b,
           lin1_W, lin1_b, lin2_W, lin2_b, lin3_W, lin3_b):
    r = edge_index[0]
    c = edge_index[1]
    rp = jnp.concatenate(
        [r, jnp.zeros((EPAD - E,), jnp.int32)])
    cp = jnp.concatenate(
        [c, jnp.full((EPAD - E,), NA - 1, jnp.int32)])
    r3 = rp.reshape(NW, NCE, KE)
    c3 = cp.reshape(NW, NCE, KE)
    batch2 = batch.astype(jnp.int32).reshape(N, 1)
    starts = jnp.searchsorted(
        batch, jnp.arange(G + 1, dtype=batch.dtype)).astype(jnp.int32)

    degp = _sc_deg(c)

    g1 = _tc_g1(x, conv1_W, degp)
    acc1 = _sc_edge(r3, c3, g1)

    xr0 = jnp.zeros((G, 2 * H), jnp.float32)
    g2, h1 = _post_mid(degp, acc1, g1, conv1_b.reshape(1, H),
                       n1_w.reshape(1, H), n1_b.reshape(1, H), batch2,
                       conv2_W)
    acc2 = _sc_edge(r3, c3, g2)
    xr1 = _pool(h1, batch2, starts, xr0)

    g3, h2 = _post_mid(degp, acc2, g2, conv2_b.reshape(1, H),
                       n2_w.reshape(1, H), n2_b.reshape(1, H), batch2,
                       conv3_W)
    acc3 = _sc_edge(r3, c3, g3)
    xr2 = _pool(h2, batch2, starts, xr1)

    lin3p = jnp.zeros((H // 2, H), jnp.float32).at[:, :C].set(lin3_W)
    lb3p = jnp.zeros((1, H), jnp.float32).at[:, :C].set(lin3_b.reshape(1, C))
    lb2p = lin2_b.reshape(1, H // 2)
    outp = _post_last(degp, acc3, g3, conv3_b.reshape(1, H),
                      n3_w.reshape(1, H), n3_b.reshape(1, H), batch2,
                      starts, xr2, lin1_W, lin2_W, lin3p,
                      lin1_b.reshape(1, H), lb2p, lb3p)
    return outp[:, :C]


# speedup vs baseline: 2.3925x; 1.0006x over previous
import functools
import jax
import jax.numpy as jnp
from jax import lax
from jax.experimental import pallas as pl
from jax.experimental.pallas import tpu as pltpu
from jax.experimental.pallas import tpu_sc as plsc

N = 10000
E = 320000
H = 128
G = 64
C = 10
EPS = 1e-5

NC = 2
NS = 16
NW = NC * NS
EPW = E // NW
KE = 80
NCE = 125
NPAIR = (NCE - 1) // 2
EPAD = NW * NCE * KE
KD = 2000
NCD = EPW // KD
NA = 10240
TR = NA // NS
NP = 10752
CH = 512


def _sc_mesh():
    return plsc.VectorSubcoreMesh(
        core_axis_name="c", subcore_axis_name="s",
        num_cores=NC, num_subcores=NS)


def _sc_deg_body(c_hbm, out_hbm, cidx, ones_v, acc):
    core = lax.axis_index("c")
    sid = lax.axis_index("s")
    w = core * NS + sid

    def zrow(i, _):
        ones_v[i, :] = jnp.zeros((16,), jnp.float32)
        return 0
    lax.fori_loop(0, TR, zrow, 0)
    pltpu.sync_copy(ones_v.at[pl.ds(0, TR)], acc.at[pl.ds(sid * TR, TR)])

    def orow(i, _):
        ones_v[i, :] = jnp.ones((16,), jnp.float32)
        return 0
    lax.fori_loop(0, KD, orow, 0)
    plsc.subcore_barrier()

    base = w * EPW

    def body(j, _):
        pltpu.sync_copy(c_hbm.at[pl.ds(base + j * KD, KD)], cidx)
        pltpu.sync_copy(ones_v, acc.at[cidx], add=True)
        return 0
    lax.fori_loop(0, NCD, body, 0)
    plsc.subcore_barrier()
    pltpu.sync_copy(acc.at[pl.ds(sid * TR, TR)],
                    out_hbm.at[core, pl.ds(sid * TR, TR)])


def _sc_edge_body(r_hbm, c_hbm, g_hbm, out_hbm, ridx, cidx,
                  rows0, rows1, acc, sem0, sem1):
    core = lax.axis_index("c")
    sid = lax.axis_index("s")
    w = core * NS + sid

    def zrow(i, _):
        for j in range(H // 16):
            rows0[i, pl.ds(j * 16, 16)] = jnp.zeros((16,), jnp.float32)
        return 0
    lax.fori_loop(0, KE, zrow, 0)
    off = 0
    while off < TR:
        step = min(KE, TR - off)
        pltpu.sync_copy(rows0.at[pl.ds(0, step)],
                        acc.at[pl.ds(sid * TR + off, step)])
        off += step
    plsc.subcore_barrier()

    pltpu.sync_copy(r_hbm.at[w, pl.ds(0, NCE)], ridx)
    pltpu.sync_copy(c_hbm.at[w, pl.ds(0, NCE)], cidx)

    def gather(j, buf, sem):
        pltpu.async_copy(g_hbm.at[ridx.at[j]], buf, sem)

    def gwait(buf, sem):
        pltpu.make_async_copy(g_hbm.at[ridx.at[0]], buf, sem).wait()

    gather(0, rows0, sem0)

    def body(p, _):
        j0 = 2 * p
        gwait(rows0, sem0)
        gather(j0 + 1, rows1, sem1)
        pltpu.sync_copy(rows0, acc.at[cidx.at[j0]], add=True)
        gwait(rows1, sem1)
        gather(j0 + 2, rows0, sem0)
        pltpu.sync_copy(rows1, acc.at[cidx.at[j0 + 1]], add=True)
        return 0
    lax.fori_loop(0, NPAIR, body, 0)
    gwait(rows0, sem0)
    pltpu.sync_copy(rows0, acc.at[cidx.at[NCE - 1]], add=True)
    plsc.subcore_barrier()
    pltpu.sync_copy(acc.at[pl.ds(sid * TR, TR)],
                    out_hbm.at[core, pl.ds(sid * TR, TR)])


@functools.cache
def _get_sc_deg():
    return pl.kernel(
        _sc_deg_body,
        out_type=jax.ShapeDtypeStruct((NC, NA, 16), jnp.float32),
        mesh=_sc_mesh(),
        compiler_params=pltpu.CompilerParams(use_tc_tiling_on_sc=False),
        scratch_types=[
            pltpu.VMEM((KD,), jnp.int32),
            pltpu.VMEM((KD, 16), jnp.float32),
            pltpu.VMEM_SHARED((NA, 16), jnp.float32),
        ],
    )


@functools.cache
def _get_sc_edge():
    return pl.kernel(
        _sc_edge_body,
        out_type=jax.ShapeDtypeStruct((NC, NA, H), jnp.float32),
        mesh=_sc_mesh(),
        compiler_params=pltpu.CompilerParams(use_tc_tiling_on_sc=False),
        scratch_types=[
            pltpu.VMEM((NCE, KE), jnp.int32),
            pltpu.VMEM((NCE, KE), jnp.int32),
            pltpu.VMEM((KE, H), jnp.float32),
            pltpu.VMEM((KE, H), jnp.float32),
            pltpu.VMEM_SHARED((NA, H), jnp.float32),
            pltpu.SemaphoreType.DMA,
            pltpu.SemaphoreType.DMA,
        ],
    )


def _sc_deg(c):
    return _get_sc_deg()(c)


def _sc_edge(r3, c3, g):
    return _get_sc_edge()(r3, c3, g)


_HI = lax.Precision.DEFAULT
_LO = lax.Precision.DEFAULT


def _dinv_from(dp_ref):
    d0 = dp_ref[0]
    d1 = dp_ref[1]
    return lax.rsqrt(1.0 + d0[:N, 0:1] + d1[:N, 0:1])


def _tc_mm_body(x_ref, w_ref, o_ref):
    o_ref[...] = lax.dot(x_ref[...], w_ref[...], precision=_HI)


def _tc_scale_body(hw_ref, dp_ref, g_ref):
    g_ref[...] = _dinv_from(dp_ref) * hw_ref[...]


def _mk_g1(interpret=False):
    mm = pl.pallas_call(
        _tc_mm_body,
        out_shape=jax.ShapeDtypeStruct((N, H), jnp.float32),
        interpret=interpret,
    )
    scale = pl.pallas_call(
        _tc_scale_body,
        out_shape=jax.ShapeDtypeStruct((N, H), jnp.float32),
        interpret=interpret,
    )

    def g1(x, w, dp):
        return scale(mm(x, w), dp)
    return g1


_tc_g1 = _mk_g1()


def _leaky(v):
    return jnp.where(v > 0, v, 0.01 * v)


RB = 1000
NRB = N // RB


def _ln_stats(dp_ref, accp_ref, g_ref, b_ref, batch_ref, hs_ref, blk_dinv, blk_oh):
    def blk1(i, carry):
        s1, s2, cnt = carry
        r0 = pl.multiple_of(i * RB, 8)
        a = accp_ref[0, pl.ds(r0, RB), :] + accp_ref[1, pl.ds(r0, RB), :]
        out = blk_dinv(i) * (a + g_ref[pl.ds(r0, RB), :]) + b_ref[...]
        hs_ref[pl.ds(r0, RB), :] = out
        oh = blk_oh(i)
        dn = (((0,), (0,)), ((), ()))
        s1 = s1 + lax.dot_general(oh, out, dn, precision=_LO)
        s2 = s2 + lax.dot_general(oh, out * out, dn, precision=_LO)
        cnt = cnt + lax.dot_general(oh, jnp.ones((RB, 1), jnp.float32), dn,
                                    precision=_LO)
        return s1, s2, cnt

    z = jnp.zeros((G, H), jnp.float32)
    s1, s2, cnt = lax.fori_loop(
        0, NRB, blk1, (z, z, jnp.zeros((G, 1), jnp.float32)))
    cntf = jnp.maximum(cnt * H, 1.0)
    mean = jnp.sum(s1, axis=1, keepdims=True) / cntf
    sq = jnp.sum(s2, axis=1, keepdims=True) / cntf
    var = jnp.maximum(sq - mean * mean, 0.0)
    rsig = lax.rsqrt(var + EPS)
    return jnp.concatenate([mean, rsig], axis=1), cnt


def _mk_blk_helpers(dp_ref, batch_ref):
    def blk_dinv(i):
        r0 = pl.multiple_of(i * RB, 8)
        d0 = dp_ref[0, pl.ds(r0, RB), :]
        d1 = dp_ref[1, pl.ds(r0, RB), :]
        return lax.rsqrt(1.0 + d0[:, 0:1] + d1[:, 0:1])

    def blk_oh(i):
        r0 = pl.multiple_of(i * RB, 8)
        bb = batch_ref[pl.ds(r0, RB), :]
        return (bb == lax.broadcasted_iota(jnp.int32, (RB, G), 1)
                ).astype(jnp.float32)
    return blk_dinv, blk_oh


def _gmp_from(hs_ref, gmp_ref, starts_ref):
    def graph_body(gi, _):
        start = starts_ref[gi]
        num = starts_ref[gi + 1] - start
        nch = (num + CH - 1) // CH

        def chunk_body(kk, m):
            astart = ((start + kk * CH) // 8) * 8
            astart = pl.multiple_of(astart, 8)
            win = hs_ref[pl.ds(astart, CH + 8), :]
            ridx = astart + lax.broadcasted_iota(jnp.int32, (CH + 8, 1), 0)
            lo = start + kk * CH
            hi = start + jnp.minimum((kk + 1) * CH, num)
            valid = (ridx >= lo) & (ridx < hi)
            vals = jnp.where(valid, win, -jnp.inf)
            return jnp.maximum(m, jnp.max(vals, axis=0, keepdims=True))

        m = lax.fori_loop(0, nch, chunk_body, jnp.full((1, H), -jnp.inf))
        m = jnp.where(num > 0, m, jnp.zeros((1, H), jnp.float32))
        gmp_ref[pl.ds(gi, 1), :] = m
        return 0
    lax.fori_loop(0, G, graph_body, 0)


def _post_mid_body(dp_ref, accp_ref, g_ref, b_ref, nw_ref, nb_ref,
                   batch_ref, wa_ref, o0_ref, o1_ref, hs_ref):
    blk_dinv, blk_oh = _mk_blk_helpers(dp_ref, batch_ref)
    stats, _ = _ln_stats(dp_ref, accp_ref, g_ref, b_ref, batch_ref, hs_ref,
                         blk_dinv, blk_oh)

    def blk2(i, _):
        r0 = pl.multiple_of(i * RB, 8)
        out = hs_ref[pl.ds(r0, RB), :]
        oh = blk_oh(i)
        nst = lax.dot_general(oh, stats, (((1,), (0,)), ((), ())),
                              precision=_LO)
        xn = (out - nst[:, 0:1]) * nst[:, 1:2] * nw_ref[...] + nb_ref[...]
        h = _leaky(xn)
        o1_ref[pl.ds(r0, RB), :] = h
        o0_ref[pl.ds(r0, RB), :] = blk_dinv(i) * lax.dot(
            h, wa_ref[...], precision=_HI)
        return 0
    lax.fori_loop(0, NRB, blk2, 0)


def _pool_body(h_ref, batch_ref, starts_ref, xr_ref, o_ref, hs_ref, gmp_ref):
    _, blk_oh = _mk_blk_helpers(None, batch_ref)

    def blk(i, carry):
        gs, cnt = carry
        r0 = pl.multiple_of(i * RB, 8)
        h = h_ref[pl.ds(r0, RB), :]
        hs_ref[pl.ds(r0, RB), :] = h
        oh = blk_oh(i)
        dn = (((0,), (0,)), ((), ()))
        gs = gs + lax.dot_general(oh, h, dn, precision=_LO)
        cnt = cnt + lax.dot_general(oh, jnp.ones((RB, 1), jnp.float32), dn,
                                    precision=_LO)
        return gs, cnt

    gs, cnt = lax.fori_loop(
        0, NRB, blk,
        (jnp.zeros((G, H), jnp.float32), jnp.zeros((G, 1), jnp.float32)))
    gap = gs / jnp.maximum(cnt, 1.0)
    _gmp_from(hs_ref, gmp_ref, starts_ref)
    o_ref[...] = jnp.concatenate([gmp_ref[...], gap], axis=1) + xr_ref[...]


def _post_last_body(dp_ref, accp_ref, g_ref, b_ref, nw_ref, nb_ref,
                    batch_ref, starts_ref, xr_ref, wa_ref, wb_ref, wc_ref,
                    ba_ref, bb_ref, bc_ref, o0_ref, hs_ref, gmp_ref):
    blk_dinv, blk_oh = _mk_blk_helpers(dp_ref, batch_ref)
    stats, cnt = _ln_stats(dp_ref, accp_ref, g_ref, b_ref, batch_ref, hs_ref,
                           blk_dinv, blk_oh)

    def blk2(i, gs):
        r0 = pl.multiple_of(i * RB, 8)
        out = hs_ref[pl.ds(r0, RB), :]
        oh = blk_oh(i)
        nst = lax.dot_general(oh, stats, (((1,), (0,)), ((), ())),
                              precision=_LO)
        xn = (out - nst[:, 0:1]) * nst[:, 1:2] * nw_ref[...] + nb_ref[...]
        h = _leaky(xn)
        hs_ref[pl.ds(r0, RB), :] = h
        return gs + lax.dot_general(oh, h, (((0,), (0,)), ((), ())),
                                    precision=_LO)

    gs = lax.fori_loop(0, NRB, blk2, jnp.zeros((G, H), jnp.float32))
    gap = gs / jnp.maximum(cnt, 1.0)
    _gmp_from(hs_ref, gmp_ref, starts_ref)
    xp = jnp.concatenate([gmp_ref[...], gap], axis=1) + xr_ref[...]

    z1 = _leaky(lax.dot(xp, wa_ref[...], precision=_HI) + ba_ref[...])
    z2 = _leaky(lax.dot(z1, wb_ref[...], precision=_HI) + bb_ref[...])
    logits = lax.dot(z2, wc_ref[...], precision=_HI) + bc_ref[...]
    colid = lax.broadcasted_iota(jnp.int32, (G, H), 1)
    lmask = colid < C
    lw = jnp.where(lmask, logits, -jnp.inf)
    mx = jnp.max(lw, axis=1, keepdims=True)
    ex = jnp.where(lmask, jnp.exp(lw - mx), 0.0)
    lse = jnp.log(jnp.sum(ex, axis=1, keepdims=True)) + mx
    o0_ref[...] = logits - lse


def _mk_post_mid(interpret=False):
    return pl.pallas_call(
        _post_mid_body,
        out_shape=[jax.ShapeDtypeStruct((N, H), jnp.float32),
                   jax.ShapeDtypeStruct((N, H), jnp.float32)],
        in_specs=[pl.BlockSpec(memory_space=pltpu.VMEM) for _ in range(8)],
        scratch_shapes=[pltpu.VMEM((NP, H), jnp.float32)],
        interpret=interpret,
    )


def _mk_pool(interpret=False):
    in_specs = [pl.BlockSpec(memory_space=pltpu.VMEM) for _ in range(4)]
    in_specs[2] = pl.BlockSpec(memory_space=pltpu.SMEM)
    return pl.pallas_call(
        _pool_body,
        out_shape=jax.ShapeDtypeStruct((G, 2 * H), jnp.float32),
        in_specs=in_specs,
        scratch_shapes=[pltpu.VMEM((NP, H), jnp.float32),
                        pltpu.VMEM((G, H), jnp.float32)],
        interpret=interpret,
    )


def _mk_post_last(interpret=False):
    in_specs = [pl.BlockSpec(memory_space=pltpu.VMEM) for _ in range(15)]
    in_specs[7] = pl.BlockSpec(memory_space=pltpu.SMEM)
    return pl.pallas_call(
        _post_last_body,
        out_shape=jax.ShapeDtypeStruct((G, H), jnp.float32),
        in_specs=in_specs,
        scratch_shapes=[pltpu.VMEM((NP, H), jnp.float32),
                        pltpu.VMEM((G, H), jnp.float32)],
        interpret=interpret,
    )


_post_mid = _mk_post_mid()
_pool = _mk_pool()
_post_last = _mk_post_last()


def kernel(x, edge_index, batch, conv1_W, conv1_b, conv2_W, conv2_b,
           conv3_W, conv3_b, n1_w, n1_b, n2_w, n2_b, n3_w, n3_b,
           lin1_W, lin1_b, lin2_W, lin2_b, lin3_W, lin3_b):
    r = edge_index[0]
    c = edge_index[1]
    rp = jnp.concatenate(
        [r, jnp.zeros((EPAD - E,), jnp.int32)])
    cp = jnp.concatenate(
        [c, jnp.full((EPAD - E,), NA - 1, jnp.int32)])
    r3 = rp.reshape(NW, NCE, KE)
    c3 = cp.reshape(NW, NCE, KE)
    batch2 = batch.astype(jnp.int32).reshape(N, 1)
    starts = jnp.searchsorted(
        batch, jnp.arange(G + 1, dtype=batch.dtype)).astype(jnp.int32)

    degp = _sc_deg(c)

    g1 = _tc_g1(x, conv1_W, degp)
    acc1 = _sc_edge(r3, c3, g1)

    xr0 = jnp.zeros((G, 2 * H), jnp.float32)
    g2, h1 = _post_mid(degp, acc1, g1, conv1_b.reshape(1, H),
                       n1_w.reshape(1, H), n1_b.reshape(1, H), batch2,
                       conv2_W)
    acc2 = _sc_edge(r3, c3, g2)
    xr1 = _pool(h1, batch2, starts, xr0)

    g3, h2 = _post_mid(degp, acc2, g2, conv2_b.reshape(1, H),
                       n2_w.reshape(1, H), n2_b.reshape(1, H), batch2,
                       conv3_W)
    acc3 = _sc_edge(r3, c3, g3)
    xr2 = _pool(h2, batch2, starts, xr1)

    lin3p = jnp.zeros((H // 2, H), jnp.float32).at[:, :C].set(lin3_W)
    lb3p = jnp.zeros((1, H), jnp.float32).at[:, :C].set(lin3_b.reshape(1, C))
    lb2p = lin2_b.reshape(1, H // 2)
    outp = _post_last(degp, acc3, g3, conv3_b.reshape(1, H),
                      n3_w.reshape(1, H), n3_b.reshape(1, H), batch2,
                      starts, xr2, lin1_W, lin2_W, lin3p,
                      lin1_b.reshape(1, H), lb2p, lb3p)
    return outp[:, :C]
